# Initial kernel scaffold; baseline (speedup 1.0000x reference)
import jax
import jax.numpy as jnp
from jax.experimental import pallas as pl


def kernel(x, edge_index, batch, embd, W_g, b_g, W_l, b_l, W_p, b_p):
    raise NotImplementedError("write your pallas kernel here")


# speedup vs baseline: 4.9043x; 4.9043x over previous
import functools

import jax
import jax.numpy as jnp
from jax import lax
from jax.experimental import pallas as pl
from jax.experimental.pallas import tpu as pltpu
from jax.experimental.pallas import tpu_sc as plsc

N_NODES = 10000
N_EDGES = 320000
DIM = 128
VOCAB_PAD = 128
N_GRAPHS = 512

NC = 2
NS = 16
NW = NC * NS

EPT = N_EDGES // NW
ECHUNK = 80
ENCHUNK = EPT // ECHUNK

RCHUNK = 80
NRCHUNK = N_NODES // RCHUNK

NCHUNK_SEG = N_NODES // ECHUNK
SEG_ROWS_PER_TILE = N_GRAPHS // NS

RB = 400
NB = N_NODES // RB


def _fill_zeros(zbuf_v, nrows):
    def zf(i, _):
        for j in range(DIM // 16):
            zbuf_v[i, pl.ds(j * 16, 16)] = jnp.zeros((16,), jnp.float32)
        return 0

    lax.fori_loop(0, nrows, zf, 0)


def _edge_agg_body(hx_hbm, src_hbm, dst_hbm, out_hbm,
                   sidx_v, didx_v, rows_v, zbuf_v,
                   agg_sh, gsem):
    cid = lax.axis_index("c")
    sid = lax.axis_index("s")
    tid = cid * NS + sid

    _fill_zeros(zbuf_v, RCHUNK)
    nkr = (NRCHUNK - sid + NS - 1) // NS

    def zero_body(k, _):
        r0 = (sid + NS * k) * RCHUNK
        pltpu.sync_copy(zbuf_v, agg_sh.at[pl.ds(r0, RCHUNK)])
        return 0

    lax.fori_loop(0, nkr, zero_body, 0)
    plsc.subcore_barrier()

    ebase = tid * EPT

    def body(c, _):
        off = ebase + c * ECHUNK
        pltpu.sync_copy(src_hbm.at[pl.ds(off, ECHUNK)], sidx_v)
        pltpu.sync_copy(dst_hbm.at[pl.ds(off, ECHUNK)], didx_v)
        pltpu.async_copy(hx_hbm.at[sidx_v], rows_v, gsem).wait()
        pltpu.sync_copy(rows_v, agg_sh.at[didx_v], add=True)
        return 0

    lax.fori_loop(0, ENCHUNK, body, 0)
    plsc.subcore_barrier()

    def wb_body(k, _):
        r0 = (sid + NS * k) * RCHUNK
        pltpu.sync_copy(agg_sh.at[pl.ds(r0, RCHUNK)],
                        out_hbm.at[cid, pl.ds(r0, RCHUNK)])
        return 0

    lax.fori_loop(0, nkr, wb_body, 0)


@jax.jit
def _edge_agg(hx, src, dst):
    mesh = plsc.VectorSubcoreMesh(core_axis_name="c", subcore_axis_name="s")
    return pl.kernel(
        _edge_agg_body,
        out_type=jax.ShapeDtypeStruct((NC, N_NODES, DIM), jnp.float32),
        mesh=mesh,
        scratch_types=[
            pltpu.VMEM((ECHUNK,), jnp.int32),
            pltpu.VMEM((ECHUNK,), jnp.int32),
            pltpu.VMEM((ECHUNK, DIM), jnp.float32),
            pltpu.VMEM((RCHUNK, DIM), jnp.float32),
            pltpu.VMEM_SHARED((N_NODES, DIM), jnp.float32),
            pltpu.SemaphoreType.DMA,
        ],
    )(hx, src, dst)


def _segsum_body(h_hbm, batch_hbm, out_hbm, rows_v, bidx_v, zbuf_v,
                 seg_sh, gsem):
    cid = lax.axis_index("c")
    sid = lax.axis_index("s")
    tid = cid * NS + sid

    _fill_zeros(zbuf_v, SEG_ROWS_PER_TILE)
    pltpu.sync_copy(zbuf_v.at[pl.ds(0, SEG_ROWS_PER_TILE)],
                    seg_sh.at[pl.ds(sid * SEG_ROWS_PER_TILE, SEG_ROWS_PER_TILE)])
    plsc.subcore_barrier()

    nk = (NCHUNK_SEG - tid + NW - 1) // NW

    def body(k, _):
        base = (tid + NW * k) * ECHUNK
        pltpu.sync_copy(h_hbm.at[pl.ds(base, ECHUNK)], rows_v)
        pltpu.sync_copy(batch_hbm.at[pl.ds(base, ECHUNK)], bidx_v)
        pltpu.sync_copy(rows_v, seg_sh.at[bidx_v], add=True)
        return 0

    lax.fori_loop(0, nk, body, 0)
    plsc.subcore_barrier()

    pltpu.sync_copy(seg_sh.at[pl.ds(sid * SEG_ROWS_PER_TILE, SEG_ROWS_PER_TILE)],
                    out_hbm.at[cid, pl.ds(sid * SEG_ROWS_PER_TILE, SEG_ROWS_PER_TILE)])


@jax.jit
def _segsum(h, batch):
    mesh = plsc.VectorSubcoreMesh(core_axis_name="c", subcore_axis_name="s")
    return pl.kernel(
        _segsum_body,
        out_type=jax.ShapeDtypeStruct((NC, N_GRAPHS, DIM), jnp.float32),
        mesh=mesh,
        scratch_types=[
            pltpu.VMEM((ECHUNK, DIM), jnp.float32),
            pltpu.VMEM((ECHUNK,), jnp.int32),
            pltpu.VMEM((SEG_ROWS_PER_TILE, DIM), jnp.float32),
            pltpu.VMEM_SHARED((N_GRAPHS, DIM), jnp.float32),
            pltpu.SemaphoreType.DMA,
        ],
    )(h, batch)


def _embed_lin_kernel(x_ref, embd_ref, w_ref, b_ref, h_ref, hx_ref):
    xb = x_ref[0, 0, :]
    iota = lax.broadcasted_iota(jnp.int32, (RB, VOCAB_PAD), 1)
    oh = (xb[:, None] == iota).astype(jnp.float32)
    h = jnp.dot(oh, embd_ref[...], preferred_element_type=jnp.float32)
    h_ref[...] = h
    hx = jnp.dot(h, w_ref[...], preferred_element_type=jnp.float32) + b_ref[...]
    hx_ref[...] = jnp.maximum(hx, 0.0)


@jax.jit
def _embed_lin(x3, embd_p, w, b):
    return pl.pallas_call(
        _embed_lin_kernel,
        grid=(NB,),
        in_specs=[
            pl.BlockSpec((1, 1, RB), lambda i: (i, 0, 0)),
            pl.BlockSpec((VOCAB_PAD, DIM), lambda i: (0, 0)),
            pl.BlockSpec((DIM, DIM), lambda i: (0, 0)),
            pl.BlockSpec((1, DIM), lambda i: (0, 0)),
        ],
        out_specs=[
            pl.BlockSpec((RB, DIM), lambda i: (i, 0)),
            pl.BlockSpec((RB, DIM), lambda i: (i, 0)),
        ],
        out_shape=[
            jax.ShapeDtypeStruct((N_NODES, DIM), jnp.float32),
            jax.ShapeDtypeStruct((N_NODES, DIM), jnp.float32),
        ],
    )(x3, embd_p, w, b)


def _layer_kernel(agg_ref, h_ref, w_ref, b_ref, hn_ref, hx_ref):
    s = agg_ref[0] + agg_ref[1] + h_ref[...]
    ss = jnp.sum(s * s, axis=1, keepdims=True)
    nrm = jnp.maximum(jnp.sqrt(ss), 1e-12)
    hn = s / nrm
    hn_ref[...] = hn
    hx = jnp.dot(hn, w_ref[...], preferred_element_type=jnp.float32) + b_ref[...]
    hx_ref[...] = jnp.maximum(hx, 0.0)


@jax.jit
def _layer(agg, h, w, b):
    return pl.pallas_call(
        _layer_kernel,
        grid=(NB,),
        in_specs=[
            pl.BlockSpec((NC, RB, DIM), lambda i: (0, i, 0)),
            pl.BlockSpec((RB, DIM), lambda i: (i, 0)),
            pl.BlockSpec((DIM, DIM), lambda i: (0, 0)),
            pl.BlockSpec((1, DIM), lambda i: (0, 0)),
        ],
        out_specs=[
            pl.BlockSpec((RB, DIM), lambda i: (i, 0)),
            pl.BlockSpec((RB, DIM), lambda i: (i, 0)),
        ],
        out_shape=[
            jax.ShapeDtypeStruct((N_NODES, DIM), jnp.float32),
            jax.ShapeDtypeStruct((N_NODES, DIM), jnp.float32),
        ],
    )(agg, h, w, b)


def _final_norm_kernel(agg_ref, h_ref, hn_ref):
    s = agg_ref[0] + agg_ref[1] + h_ref[...]
    ss = jnp.sum(s * s, axis=1, keepdims=True)
    nrm = jnp.maximum(jnp.sqrt(ss), 1e-12)
    hn_ref[...] = s / nrm


@jax.jit
def _final_norm(agg, h):
    return pl.pallas_call(
        _final_norm_kernel,
        grid=(NB,),
        in_specs=[
            pl.BlockSpec((NC, RB, DIM), lambda i: (0, i, 0)),
            pl.BlockSpec((RB, DIM), lambda i: (i, 0)),
        ],
        out_specs=pl.BlockSpec((RB, DIM), lambda i: (i, 0)),
        out_shape=jax.ShapeDtypeStruct((N_NODES, DIM), jnp.float32),
    )(agg, h)


def _readout_kernel(seg_ref, wl_ref, bl_ref, wp_ref, bp_ref, out_ref):
    m = seg_ref[0] + seg_ref[1]
    for i in range(2):
        m = jnp.dot(m, wl_ref[i], preferred_element_type=jnp.float32)
        m = jnp.maximum(m + bl_ref[i:i + 1, :], 0.0)
    out = jnp.dot(m, wp_ref[...], preferred_element_type=jnp.float32)
    out_ref[...] = out + bp_ref[...]


@jax.jit
def _readout(seg, wl, bl, wp, bp):
    return pl.pallas_call(
        _readout_kernel,
        out_shape=jax.ShapeDtypeStruct((N_GRAPHS, 1), jnp.float32),
    )(seg, wl, bl, wp, bp)


def kernel(x, edge_index, batch, embd, W_g, b_g, W_l, b_l, W_p, b_p):
    x3 = x.astype(jnp.int32).reshape(NB, 1, RB)
    src = edge_index[0].astype(jnp.int32)
    dst = edge_index[1].astype(jnp.int32)
    batch = batch.astype(jnp.int32)
    embd_p = jnp.pad(embd, ((0, VOCAB_PAD - embd.shape[0]), (0, 0)))

    h, hx = _embed_lin(x3, embd_p, W_g[0], b_g[0].reshape(1, DIM))
    for m in range(3):
        agg = _edge_agg(hx, src, dst)
        if m < 2:
            h, hx = _layer(agg, h, W_g[m + 1], b_g[m + 1].reshape(1, DIM))
        else:
            h = _final_norm(agg, h)

    seg = _segsum(h, batch)
    props = _readout(seg, W_l, b_l, W_p, b_p.reshape(1, 1))
    return props.reshape(N_GRAPHS)


# speedup vs baseline: 8.9815x; 1.8314x over previous
import functools

import jax
import jax.numpy as jnp
from jax import lax
from jax.experimental import pallas as pl
from jax.experimental.pallas import tpu as pltpu
from jax.experimental.pallas import tpu_sc as plsc

N_NODES = 10000
N_EDGES = 320000
DIM = 128
VOCAB_PAD = 128
N_GRAPHS = 512

NC = 2
NS = 16
NW = NC * NS

EPT = N_EDGES // NW
ECHUNK = 80
ENCHUNK = EPT // ECHUNK

RCHUNK = 80
NRCHUNK = N_NODES // RCHUNK

NCHUNK_SEG = N_NODES // ECHUNK
SEG_ROWS_PER_TILE = N_GRAPHS // NS

RB = 400
NB = N_NODES // RB


def _fill_zeros(zbuf_v, nrows):
    def zf(i, _):
        for j in range(DIM // 16):
            zbuf_v[i, pl.ds(j * 16, 16)] = jnp.zeros((16,), jnp.float32)
        return 0

    lax.fori_loop(0, nrows, zf, 0)


def _edge_agg_body(hx_hbm, idx_hbm, out_hbm,
                   idx0_v, idx1_v, rows0_v, rows1_v, zbuf_v,
                   agg_sh, isem0, isem1, gsem0, gsem1):
    cid = lax.axis_index("c")
    sid = lax.axis_index("s")
    tid = cid * NS + sid

    _fill_zeros(zbuf_v, RCHUNK)
    nkr = (NRCHUNK - sid + NS - 1) // NS

    def zero_body(k, _):
        r0 = (sid + NS * k) * RCHUNK
        pltpu.sync_copy(zbuf_v, agg_sh.at[pl.ds(r0, RCHUNK)])
        return 0

    lax.fori_loop(0, nkr, zero_body, 0)
    plsc.subcore_barrier()

    cbase = tid * ENCHUNK

    def gather_start(idx_v, rows_v, isem, gsem, c):
        pltpu.make_async_copy(idx_hbm.at[cbase + c], idx_v, isem).wait()
        pltpu.async_copy(hx_hbm.at[idx_v.at[0]], rows_v, gsem)

    def scatter_prev(idx_v, rows_v, gsem):
        pltpu.make_async_copy(hx_hbm.at[idx_v.at[0]], rows_v, gsem).wait()
        pltpu.sync_copy(rows_v, agg_sh.at[idx_v.at[1]], add=True)

    pltpu.async_copy(idx_hbm.at[cbase], idx0_v, isem0)

    def body(i, _):
        even = (i % 2) == 0

        @pl.when(even)
        def _():
            gather_start(idx0_v, rows0_v, isem0, gsem0, i)

        @pl.when(jnp.logical_not(even))
        def _():
            gather_start(idx1_v, rows1_v, isem1, gsem1, i)

        @pl.when(jnp.logical_and(i > 0, even))
        def _():
            scatter_prev(idx1_v, rows1_v, gsem1)

        @pl.when(jnp.logical_not(even))
        def _():
            scatter_prev(idx0_v, rows0_v, gsem0)

        @pl.when(jnp.logical_and(i + 1 < ENCHUNK, even))
        def _():
            pltpu.async_copy(idx_hbm.at[cbase + i + 1], idx1_v, isem1)

        @pl.when(jnp.logical_and(i + 1 < ENCHUNK, jnp.logical_not(even)))
        def _():
            pltpu.async_copy(idx_hbm.at[cbase + i + 1], idx0_v, isem0)

        return 0

    lax.fori_loop(0, ENCHUNK, body, 0)
    scatter_prev(idx0_v, rows0_v, gsem0)
    plsc.subcore_barrier()

    def wb_body(k, _):
        r0 = (sid + NS * k) * RCHUNK
        pltpu.sync_copy(agg_sh.at[pl.ds(r0, RCHUNK)],
                        out_hbm.at[cid, pl.ds(r0, RCHUNK)])
        return 0

    lax.fori_loop(0, nkr, wb_body, 0)


@jax.jit
def _edge_agg(hx, idx):
    mesh = plsc.VectorSubcoreMesh(core_axis_name="c", subcore_axis_name="s")
    return pl.kernel(
        _edge_agg_body,
        out_type=jax.ShapeDtypeStruct((NC, N_NODES, DIM), jnp.float32),
        mesh=mesh,
        scratch_types=[
            pltpu.VMEM((2, ECHUNK), jnp.int32),
            pltpu.VMEM((2, ECHUNK), jnp.int32),
            pltpu.VMEM((ECHUNK, DIM), jnp.float32),
            pltpu.VMEM((ECHUNK, DIM), jnp.float32),
            pltpu.VMEM((RCHUNK, DIM), jnp.float32),
            pltpu.VMEM_SHARED((N_NODES, DIM), jnp.float32),
            pltpu.SemaphoreType.DMA,
            pltpu.SemaphoreType.DMA,
            pltpu.SemaphoreType.DMA,
            pltpu.SemaphoreType.DMA,
        ],
    )(hx, idx)


def _segsum_body(h_hbm, batch_hbm, out_hbm, rows_v, bidx_v, zbuf_v,
                 seg_sh, gsem):
    cid = lax.axis_index("c")
    sid = lax.axis_index("s")
    tid = cid * NS + sid

    _fill_zeros(zbuf_v, SEG_ROWS_PER_TILE)
    pltpu.sync_copy(zbuf_v.at[pl.ds(0, SEG_ROWS_PER_TILE)],
                    seg_sh.at[pl.ds(sid * SEG_ROWS_PER_TILE, SEG_ROWS_PER_TILE)])
    plsc.subcore_barrier()

    nk = (NCHUNK_SEG - tid + NW - 1) // NW

    def body(k, _):
        base = (tid + NW * k) * ECHUNK
        pltpu.sync_copy(h_hbm.at[pl.ds(base, ECHUNK)], rows_v)
        pltpu.sync_copy(batch_hbm.at[pl.ds(base, ECHUNK)], bidx_v)
        pltpu.sync_copy(rows_v, seg_sh.at[bidx_v], add=True)
        return 0

    lax.fori_loop(0, nk, body, 0)
    plsc.subcore_barrier()

    pltpu.sync_copy(seg_sh.at[pl.ds(sid * SEG_ROWS_PER_TILE, SEG_ROWS_PER_TILE)],
                    out_hbm.at[cid, pl.ds(sid * SEG_ROWS_PER_TILE, SEG_ROWS_PER_TILE)])


@jax.jit
def _segsum(h, batch):
    mesh = plsc.VectorSubcoreMesh(core_axis_name="c", subcore_axis_name="s")
    return pl.kernel(
        _segsum_body,
        out_type=jax.ShapeDtypeStruct((NC, N_GRAPHS, DIM), jnp.float32),
        mesh=mesh,
        scratch_types=[
            pltpu.VMEM((ECHUNK, DIM), jnp.float32),
            pltpu.VMEM((ECHUNK,), jnp.int32),
            pltpu.VMEM((SEG_ROWS_PER_TILE, DIM), jnp.float32),
            pltpu.VMEM_SHARED((N_GRAPHS, DIM), jnp.float32),
            pltpu.SemaphoreType.DMA,
        ],
    )(h, batch)


def _embed_lin_kernel(x_ref, embd_ref, w_ref, b_ref, h_ref, hx_ref):
    xb = x_ref[0, 0, :]
    iota = lax.broadcasted_iota(jnp.int32, (RB, VOCAB_PAD), 1)
    oh = (xb[:, None] == iota).astype(jnp.float32)
    h = jnp.dot(oh, embd_ref[...], preferred_element_type=jnp.float32)
    h_ref[...] = h
    hx = jnp.dot(h, w_ref[...], preferred_element_type=jnp.float32) + b_ref[...]
    hx_ref[...] = jnp.maximum(hx, 0.0)


@jax.jit
def _embed_lin(x3, embd_p, w, b):
    return pl.pallas_call(
        _embed_lin_kernel,
        grid=(NB,),
        in_specs=[
            pl.BlockSpec((1, 1, RB), lambda i: (i, 0, 0)),
            pl.BlockSpec((VOCAB_PAD, DIM), lambda i: (0, 0)),
            pl.BlockSpec((DIM, DIM), lambda i: (0, 0)),
            pl.BlockSpec((1, DIM), lambda i: (0, 0)),
        ],
        out_specs=[
            pl.BlockSpec((RB, DIM), lambda i: (i, 0)),
            pl.BlockSpec((RB, DIM), lambda i: (i, 0)),
        ],
        out_shape=[
            jax.ShapeDtypeStruct((N_NODES, DIM), jnp.float32),
            jax.ShapeDtypeStruct((N_NODES, DIM), jnp.float32),
        ],
    )(x3, embd_p, w, b)


def _layer_kernel(agg_ref, h_ref, w_ref, b_ref, hn_ref, hx_ref):
    s = agg_ref[0] + agg_ref[1] + h_ref[...]
    ss = jnp.sum(s * s, axis=1, keepdims=True)
    nrm = jnp.maximum(jnp.sqrt(ss), 1e-12)
    hn = s / nrm
    hn_ref[...] = hn
    hx = jnp.dot(hn, w_ref[...], preferred_element_type=jnp.float32) + b_ref[...]
    hx_ref[...] = jnp.maximum(hx, 0.0)


@jax.jit
def _layer(agg, h, w, b):
    return pl.pallas_call(
        _layer_kernel,
        grid=(NB,),
        in_specs=[
            pl.BlockSpec((NC, RB, DIM), lambda i: (0, i, 0)),
            pl.BlockSpec((RB, DIM), lambda i: (i, 0)),
            pl.BlockSpec((DIM, DIM), lambda i: (0, 0)),
            pl.BlockSpec((1, DIM), lambda i: (0, 0)),
        ],
        out_specs=[
            pl.BlockSpec((RB, DIM), lambda i: (i, 0)),
            pl.BlockSpec((RB, DIM), lambda i: (i, 0)),
        ],
        out_shape=[
            jax.ShapeDtypeStruct((N_NODES, DIM), jnp.float32),
            jax.ShapeDtypeStruct((N_NODES, DIM), jnp.float32),
        ],
    )(agg, h, w, b)


def _final_norm_kernel(agg_ref, h_ref, hn_ref):
    s = agg_ref[0] + agg_ref[1] + h_ref[...]
    ss = jnp.sum(s * s, axis=1, keepdims=True)
    nrm = jnp.maximum(jnp.sqrt(ss), 1e-12)
    hn_ref[...] = s / nrm


@jax.jit
def _final_norm(agg, h):
    return pl.pallas_call(
        _final_norm_kernel,
        grid=(NB,),
        in_specs=[
            pl.BlockSpec((NC, RB, DIM), lambda i: (0, i, 0)),
            pl.BlockSpec((RB, DIM), lambda i: (i, 0)),
        ],
        out_specs=pl.BlockSpec((RB, DIM), lambda i: (i, 0)),
        out_shape=jax.ShapeDtypeStruct((N_NODES, DIM), jnp.float32),
    )(agg, h)


def _readout_kernel(seg_ref, wl_ref, bl_ref, wp_ref, bp_ref, out_ref):
    m = seg_ref[0] + seg_ref[1]
    for i in range(2):
        m = jnp.dot(m, wl_ref[i], preferred_element_type=jnp.float32)
        m = jnp.maximum(m + bl_ref[i:i + 1, :], 0.0)
    out = jnp.dot(m, wp_ref[...], preferred_element_type=jnp.float32)
    out_ref[...] = out + bp_ref[...]


@jax.jit
def _readout(seg, wl, bl, wp, bp):
    return pl.pallas_call(
        _readout_kernel,
        out_shape=jax.ShapeDtypeStruct((N_GRAPHS, 1), jnp.float32),
    )(seg, wl, bl, wp, bp)


def kernel(x, edge_index, batch, embd, W_g, b_g, W_l, b_l, W_p, b_p):
    x3 = x.astype(jnp.int32).reshape(NB, 1, RB)
    ei = edge_index.astype(jnp.int32)
    idx = jnp.stack(
        [ei[0].reshape(N_EDGES // ECHUNK, ECHUNK),
         ei[1].reshape(N_EDGES // ECHUNK, ECHUNK)], axis=1)
    batch = batch.astype(jnp.int32)
    embd_p = jnp.pad(embd, ((0, VOCAB_PAD - embd.shape[0]), (0, 0)))

    h, hx = _embed_lin(x3, embd_p, W_g[0], b_g[0].reshape(1, DIM))
    for m in range(3):
        agg = _edge_agg(hx, idx)
        if m < 2:
            h, hx = _layer(agg, h, W_g[m + 1], b_g[m + 1].reshape(1, DIM))
        else:
            h = _final_norm(agg, h)

    seg = _segsum(h, batch)
    props = _readout(seg, W_l, b_l, W_p, b_p.reshape(1, 1))
    return props.reshape(N_GRAPHS)


# speedup vs baseline: 11.2162x; 1.2488x over previous
import functools

import jax
import jax.numpy as jnp
from jax import lax
from jax.experimental import pallas as pl
from jax.experimental.pallas import tpu as pltpu
from jax.experimental.pallas import tpu_sc as plsc

N_NODES = 10000
N_EDGES = 320000
DIM = 128
VOCAB_PAD = 128
N_GRAPHS = 512

NC = 2
NS = 16
NW = NC * NS

EPT = N_EDGES // NW
ECHUNK = 125
ENCHUNK = EPT // ECHUNK

RCHUNK = 40
NRCHUNK = N_NODES // RCHUNK

SEGCHUNK = 80
NCHUNK_SEG = N_NODES // SEGCHUNK
SEG_ROWS_PER_TILE = N_GRAPHS // NS

RB = 400
NB = N_NODES // RB


def _fill_zeros(zbuf_v, nrows):
    def zf(i, _):
        for j in range(DIM // 16):
            zbuf_v[i, pl.ds(j * 16, 16)] = jnp.zeros((16,), jnp.float32)
        return 0

    lax.fori_loop(0, nrows, zf, 0)


def _edge_agg_body(hx_hbm, idx_hbm, out_hbm,
                   idx0_v, idx1_v, idx2_v, idx3_v, rows0_v, rows1_v, zbuf_v,
                   agg_sh, isem0, isem1, isem2, isem3, gsem0, gsem1):
    cid = lax.axis_index("c")
    sid = lax.axis_index("s")
    tid = cid * NS + sid

    IDX = [idx0_v, idx1_v, idx2_v, idx3_v]
    ROWS = [rows0_v, rows1_v]
    ISEM = [isem0, isem1, isem2, isem3]
    GSEM = [gsem0, gsem1]

    _fill_zeros(zbuf_v, RCHUNK)
    nkr = (NRCHUNK - sid + NS - 1) // NS

    def zero_body(k, _):
        r0 = (sid + NS * k) * RCHUNK
        pltpu.sync_copy(zbuf_v, agg_sh.at[pl.ds(r0, RCHUNK)])
        return 0

    lax.fori_loop(0, nkr, zero_body, 0)
    plsc.subcore_barrier()

    cbase = tid * ENCHUNK

    def idx_load(c, q):
        pltpu.async_copy(idx_hbm.at[cbase + c], IDX[q], ISEM[q])

    def gather(c, q, r):
        pltpu.make_async_copy(idx_hbm.at[cbase + c], IDX[q], ISEM[q]).wait()
        pltpu.async_copy(hx_hbm.at[IDX[q].at[0]], ROWS[r], GSEM[r])

    def scatter(q, r):
        pltpu.make_async_copy(hx_hbm.at[IDX[q].at[0]], ROWS[r], GSEM[r]).wait()
        pltpu.sync_copy(ROWS[r], agg_sh.at[IDX[q].at[1]], add=True)

    idx_load(0, 0)
    idx_load(1, 1)
    idx_load(2, 2)
    gather(0, 0, 0)
    idx_load(3, 3)
    gather(1, 1, 1)
    scatter(0, 0)
    idx_load(4, 0)
    gather(2, 2, 0)
    scatter(1, 1)
    idx_load(5, 1)
    gather(3, 3, 1)
    scatter(2, 0)
    idx_load(6, 2)

    def body(j, _):
        c = 4 * j
        gather(c, 0, 0)
        scatter(3, 1)
        idx_load(c + 3, 3)
        gather(c + 1, 1, 1)
        scatter(0, 0)
        idx_load(c + 4, 0)
        gather(c + 2, 2, 0)
        scatter(1, 1)
        idx_load(c + 5, 1)
        gather(c + 3, 3, 1)
        scatter(2, 0)
        idx_load(c + 6, 2)
        return 0

    lax.fori_loop(1, ENCHUNK // 4 - 1, body, 0)

    cl = ENCHUNK - 4
    gather(cl, 0, 0)
    scatter(3, 1)
    idx_load(cl + 3, 3)
    gather(cl + 1, 1, 1)
    scatter(0, 0)
    gather(cl + 2, 2, 0)
    scatter(1, 1)
    gather(cl + 3, 3, 1)
    scatter(2, 0)
    scatter(3, 1)
    plsc.subcore_barrier()

    def wb_body(k, _):
        r0 = (sid + NS * k) * RCHUNK
        pltpu.sync_copy(agg_sh.at[pl.ds(r0, RCHUNK)],
                        out_hbm.at[cid, pl.ds(r0, RCHUNK)])
        return 0

    lax.fori_loop(0, nkr, wb_body, 0)


@jax.jit
def _edge_agg(hx, idx):
    mesh = plsc.VectorSubcoreMesh(core_axis_name="c", subcore_axis_name="s")
    return pl.kernel(
        _edge_agg_body,
        out_type=jax.ShapeDtypeStruct((NC, N_NODES, DIM), jnp.float32),
        mesh=mesh,
        scratch_types=(
            [pltpu.VMEM((2, ECHUNK), jnp.int32)] * 4
            + [pltpu.VMEM((ECHUNK, DIM), jnp.float32)] * 2
            + [pltpu.VMEM((RCHUNK, DIM), jnp.float32)]
            + [pltpu.VMEM_SHARED((N_NODES, DIM), jnp.float32)]
            + [pltpu.SemaphoreType.DMA] * 6
        ),
    )(hx, idx)


def _segsum_body(h_hbm, batch_hbm, out_hbm, rows_v, bidx_v, zbuf_v,
                 seg_sh, gsem):
    cid = lax.axis_index("c")
    sid = lax.axis_index("s")
    tid = cid * NS + sid

    _fill_zeros(zbuf_v, SEG_ROWS_PER_TILE)
    pltpu.sync_copy(zbuf_v.at[pl.ds(0, SEG_ROWS_PER_TILE)],
                    seg_sh.at[pl.ds(sid * SEG_ROWS_PER_TILE, SEG_ROWS_PER_TILE)])
    plsc.subcore_barrier()

    nk = (NCHUNK_SEG - tid + NW - 1) // NW

    def body(k, _):
        base = (tid + NW * k) * SEGCHUNK
        pltpu.sync_copy(h_hbm.at[pl.ds(base, SEGCHUNK)], rows_v)
        pltpu.sync_copy(batch_hbm.at[pl.ds(base, SEGCHUNK)], bidx_v)
        pltpu.sync_copy(rows_v, seg_sh.at[bidx_v], add=True)
        return 0

    lax.fori_loop(0, nk, body, 0)
    plsc.subcore_barrier()

    pltpu.sync_copy(seg_sh.at[pl.ds(sid * SEG_ROWS_PER_TILE, SEG_ROWS_PER_TILE)],
                    out_hbm.at[cid, pl.ds(sid * SEG_ROWS_PER_TILE, SEG_ROWS_PER_TILE)])


@jax.jit
def _segsum(h, batch):
    mesh = plsc.VectorSubcoreMesh(core_axis_name="c", subcore_axis_name="s")
    return pl.kernel(
        _segsum_body,
        out_type=jax.ShapeDtypeStruct((NC, N_GRAPHS, DIM), jnp.float32),
        mesh=mesh,
        scratch_types=[
            pltpu.VMEM((SEGCHUNK, DIM), jnp.float32),
            pltpu.VMEM((SEGCHUNK,), jnp.int32),
            pltpu.VMEM((SEG_ROWS_PER_TILE, DIM), jnp.float32),
            pltpu.VMEM_SHARED((N_GRAPHS, DIM), jnp.float32),
            pltpu.SemaphoreType.DMA,
        ],
    )(h, batch)


def _embed_lin_kernel(x_ref, embd_ref, w_ref, b_ref, h_ref, hx_ref):
    xb = x_ref[0, 0, :]
    iota = lax.broadcasted_iota(jnp.int32, (RB, VOCAB_PAD), 1)
    oh = (xb[:, None] == iota).astype(jnp.float32)
    h = jnp.dot(oh, embd_ref[...], preferred_element_type=jnp.float32)
    h_ref[...] = h
    hx = jnp.dot(h, w_ref[...], preferred_element_type=jnp.float32) + b_ref[...]
    hx_ref[...] = jnp.maximum(hx, 0.0)


@jax.jit
def _embed_lin(x3, embd_p, w, b):
    return pl.pallas_call(
        _embed_lin_kernel,
        grid=(NB,),
        in_specs=[
            pl.BlockSpec((1, 1, RB), lambda i: (i, 0, 0)),
            pl.BlockSpec((VOCAB_PAD, DIM), lambda i: (0, 0)),
            pl.BlockSpec((DIM, DIM), lambda i: (0, 0)),
            pl.BlockSpec((1, DIM), lambda i: (0, 0)),
        ],
        out_specs=[
            pl.BlockSpec((RB, DIM), lambda i: (i, 0)),
            pl.BlockSpec((RB, DIM), lambda i: (i, 0)),
        ],
        out_shape=[
            jax.ShapeDtypeStruct((N_NODES, DIM), jnp.float32),
            jax.ShapeDtypeStruct((N_NODES, DIM), jnp.float32),
        ],
    )(x3, embd_p, w, b)


def _layer_kernel(agg_ref, h_ref, w_ref, b_ref, hn_ref, hx_ref):
    s = agg_ref[0] + agg_ref[1] + h_ref[...]
    ss = jnp.sum(s * s, axis=1, keepdims=True)
    nrm = jnp.maximum(jnp.sqrt(ss), 1e-12)
    hn = s / nrm
    hn_ref[...] = hn
    hx = jnp.dot(hn, w_ref[...], preferred_element_type=jnp.float32) + b_ref[...]
    hx_ref[...] = jnp.maximum(hx, 0.0)


@jax.jit
def _layer(agg, h, w, b):
    return pl.pallas_call(
        _layer_kernel,
        grid=(NB,),
        in_specs=[
            pl.BlockSpec((NC, RB, DIM), lambda i: (0, i, 0)),
            pl.BlockSpec((RB, DIM), lambda i: (i, 0)),
            pl.BlockSpec((DIM, DIM), lambda i: (0, 0)),
            pl.BlockSpec((1, DIM), lambda i: (0, 0)),
        ],
        out_specs=[
            pl.BlockSpec((RB, DIM), lambda i: (i, 0)),
            pl.BlockSpec((RB, DIM), lambda i: (i, 0)),
        ],
        out_shape=[
            jax.ShapeDtypeStruct((N_NODES, DIM), jnp.float32),
            jax.ShapeDtypeStruct((N_NODES, DIM), jnp.float32),
        ],
    )(agg, h, w, b)


def _final_norm_kernel(agg_ref, h_ref, hn_ref):
    s = agg_ref[0] + agg_ref[1] + h_ref[...]
    ss = jnp.sum(s * s, axis=1, keepdims=True)
    nrm = jnp.maximum(jnp.sqrt(ss), 1e-12)
    hn_ref[...] = s / nrm


@jax.jit
def _final_norm(agg, h):
    return pl.pallas_call(
        _final_norm_kernel,
        grid=(NB,),
        in_specs=[
            pl.BlockSpec((NC, RB, DIM), lambda i: (0, i, 0)),
            pl.BlockSpec((RB, DIM), lambda i: (i, 0)),
        ],
        out_specs=pl.BlockSpec((RB, DIM), lambda i: (i, 0)),
        out_shape=jax.ShapeDtypeStruct((N_NODES, DIM), jnp.float32),
    )(agg, h)


def _readout_kernel(seg_ref, wl_ref, bl_ref, wp_ref, bp_ref, out_ref):
    m = seg_ref[0] + seg_ref[1]
    for i in range(2):
        m = jnp.dot(m, wl_ref[i], preferred_element_type=jnp.float32)
        m = jnp.maximum(m + bl_ref[i:i + 1, :], 0.0)
    out = jnp.dot(m, wp_ref[...], preferred_element_type=jnp.float32)
    out_ref[...] = out + bp_ref[...]


@jax.jit
def _readout(seg, wl, bl, wp, bp):
    return pl.pallas_call(
        _readout_kernel,
        out_shape=jax.ShapeDtypeStruct((N_GRAPHS, 1), jnp.float32),
    )(seg, wl, bl, wp, bp)


def kernel(x, edge_index, batch, embd, W_g, b_g, W_l, b_l, W_p, b_p):
    x3 = x.astype(jnp.int32).reshape(NB, 1, RB)
    ei = edge_index.astype(jnp.int32)
    idx = jnp.stack(
        [ei[0].reshape(N_EDGES // ECHUNK, ECHUNK),
         ei[1].reshape(N_EDGES // ECHUNK, ECHUNK)], axis=1)
    batch = batch.astype(jnp.int32)
    embd_p = jnp.pad(embd, ((0, VOCAB_PAD - embd.shape[0]), (0, 0)))

    h, hx = _embed_lin(x3, embd_p, W_g[0], b_g[0].reshape(1, DIM))
    for m in range(3):
        agg = _edge_agg(hx, idx)
        if m < 2:
            h, hx = _layer(agg, h, W_g[m + 1], b_g[m + 1].reshape(1, DIM))
        else:
            h = _final_norm(agg, h)

    seg = _segsum(h, batch)
    props = _readout(seg, W_l, b_l, W_p, b_p.reshape(1, 1))
    return props.reshape(N_GRAPHS)


# speedup vs baseline: 11.3826x; 1.0148x over previous
import functools

import jax
import jax.numpy as jnp
from jax import lax
from jax.experimental import pallas as pl
from jax.experimental.pallas import tpu as pltpu
from jax.experimental.pallas import tpu_sc as plsc

N_NODES = 10000
N_EDGES = 320000
DIM = 128
VOCAB_PAD = 128
N_GRAPHS = 512

NC = 2
NS = 16
NW = NC * NS

EPT = N_EDGES // NW
ECHUNK = 125
ENCHUNK = EPT // ECHUNK

RCHUNK = 40
NRCHUNK = N_NODES // RCHUNK

SEGCHUNK = 80
NCHUNK_SEG = N_NODES // SEGCHUNK
SEG_ROWS_PER_TILE = N_GRAPHS // NS

RB = 400
NB = N_NODES // RB


def _fill_zeros(zbuf_v, nrows):
    def zf(i, _):
        for j in range(DIM // 16):
            zbuf_v[i, pl.ds(j * 16, 16)] = jnp.zeros((16,), jnp.float32)
        return 0

    lax.fori_loop(0, nrows, zf, 0)


def _edge_agg_body(hx_hbm, idx_hbm, out_hbm,
                   idx0_v, idx1_v, idx2_v, idx3_v, rows0_v, rows1_v, zbuf_v,
                   agg_sh, isem0, isem1, isem2, isem3, gsem0, gsem1):
    cid = lax.axis_index("c")
    sid = lax.axis_index("s")
    tid = cid * NS + sid

    IDX = [idx0_v, idx1_v, idx2_v, idx3_v]
    ROWS = [rows0_v, rows1_v]
    ISEM = [isem0, isem1, isem2, isem3]
    GSEM = [gsem0, gsem1]

    _fill_zeros(zbuf_v, RCHUNK)
    nkr = (NRCHUNK - sid + NS - 1) // NS

    def zero_body(k, _):
        r0 = (sid + NS * k) * RCHUNK
        pltpu.sync_copy(zbuf_v, agg_sh.at[pl.ds(r0, RCHUNK)])
        return 0

    lax.fori_loop(0, nkr, zero_body, 0)
    plsc.subcore_barrier()

    cbase = tid * ENCHUNK

    def idx_load(c, q):
        pltpu.async_copy(idx_hbm.at[0, cbase + c], IDX[q].at[0], ISEM[q])
        pltpu.async_copy(idx_hbm.at[1, cbase + c], IDX[q].at[1], ISEM[q])

    def gather(c, q, r):
        pltpu.make_async_copy(idx_hbm.at[0, cbase + c], IDX[q].at[0], ISEM[q]).wait()
        pltpu.make_async_copy(idx_hbm.at[1, cbase + c], IDX[q].at[1], ISEM[q]).wait()
        pltpu.async_copy(hx_hbm.at[IDX[q].at[0]], ROWS[r], GSEM[r])

    def scatter(q, r):
        pltpu.make_async_copy(hx_hbm.at[IDX[q].at[0]], ROWS[r], GSEM[r]).wait()
        pltpu.sync_copy(ROWS[r], agg_sh.at[IDX[q].at[1]], add=True)

    idx_load(0, 0)
    idx_load(1, 1)
    idx_load(2, 2)
    gather(0, 0, 0)
    idx_load(3, 3)
    gather(1, 1, 1)
    scatter(0, 0)
    idx_load(4, 0)
    gather(2, 2, 0)
    scatter(1, 1)
    idx_load(5, 1)
    gather(3, 3, 1)
    scatter(2, 0)
    idx_load(6, 2)

    def body(j, _):
        c = 4 * j
        gather(c, 0, 0)
        scatter(3, 1)
        idx_load(c + 3, 3)
        gather(c + 1, 1, 1)
        scatter(0, 0)
        idx_load(c + 4, 0)
        gather(c + 2, 2, 0)
        scatter(1, 1)
        idx_load(c + 5, 1)
        gather(c + 3, 3, 1)
        scatter(2, 0)
        idx_load(c + 6, 2)
        return 0

    lax.fori_loop(1, ENCHUNK // 4 - 1, body, 0)

    cl = ENCHUNK - 4
    gather(cl, 0, 0)
    scatter(3, 1)
    idx_load(cl + 3, 3)
    gather(cl + 1, 1, 1)
    scatter(0, 0)
    gather(cl + 2, 2, 0)
    scatter(1, 1)
    gather(cl + 3, 3, 1)
    scatter(2, 0)
    scatter(3, 1)
    plsc.subcore_barrier()

    def wb_body(k, _):
        r0 = (sid + NS * k) * RCHUNK
        pltpu.sync_copy(agg_sh.at[pl.ds(r0, RCHUNK)],
                        out_hbm.at[cid, pl.ds(r0, RCHUNK)])
        return 0

    lax.fori_loop(0, nkr, wb_body, 0)


@jax.jit
def _edge_agg(hx, idx):
    mesh = plsc.VectorSubcoreMesh(core_axis_name="c", subcore_axis_name="s")
    return pl.kernel(
        _edge_agg_body,
        out_type=jax.ShapeDtypeStruct((NC, N_NODES, DIM), jnp.float32),
        mesh=mesh,
        scratch_types=(
            [pltpu.VMEM((2, ECHUNK), jnp.int32)] * 4
            + [pltpu.VMEM((ECHUNK, DIM), jnp.float32)] * 2
            + [pltpu.VMEM((RCHUNK, DIM), jnp.float32)]
            + [pltpu.VMEM_SHARED((N_NODES, DIM), jnp.float32)]
            + [pltpu.SemaphoreType.DMA] * 6
        ),
    )(hx, idx)


def _segsum_body(agg_hbm, h_hbm, batch_hbm, out_hbm,
                 a0_v, a1_v, s_v, bidx_v, zbuf_v, seg_sh, gsem):
    cid = lax.axis_index("c")
    sid = lax.axis_index("s")
    tid = cid * NS + sid

    _fill_zeros(zbuf_v, SEG_ROWS_PER_TILE)
    pltpu.sync_copy(zbuf_v.at[pl.ds(0, SEG_ROWS_PER_TILE)],
                    seg_sh.at[pl.ds(sid * SEG_ROWS_PER_TILE, SEG_ROWS_PER_TILE)])
    plsc.subcore_barrier()

    nk = (NCHUNK_SEG - tid + NW - 1) // NW

    def body(k, _):
        base = (tid + NW * k) * SEGCHUNK
        pltpu.sync_copy(agg_hbm.at[0, pl.ds(base, SEGCHUNK)], a0_v)
        pltpu.sync_copy(agg_hbm.at[1, pl.ds(base, SEGCHUNK)], a1_v)
        pltpu.sync_copy(h_hbm.at[pl.ds(base, SEGCHUNK)], s_v)
        pltpu.sync_copy(batch_hbm.at[pl.ds(base, SEGCHUNK)], bidx_v)

        def row(r, _):
            ss = jnp.zeros((16,), jnp.float32)
            sl = []
            for v in range(DIM // 16):
                x = (a0_v[r, pl.ds(16 * v, 16)] + a1_v[r, pl.ds(16 * v, 16)]
                     + s_v[r, pl.ds(16 * v, 16)])
                sl.append(x)
                ss = ss + x * x
            lane = lax.iota(jnp.int32, 16)
            for sh in (1, 2, 4, 8):
                ss = ss + ss.at[jnp.bitwise_xor(lane, sh)].get(
                    mode="promise_in_bounds")
            tot = jnp.maximum(ss[0], jnp.float32(1e-24))
            i = lax.bitcast_convert_type(tot, jnp.int32)
            g = lax.bitcast_convert_type(
                jnp.int32(0x5F3759DF) - lax.shift_right_arithmetic(i, 1),
                jnp.float32)
            for _ in range(3):
                g = g * (1.5 - 0.5 * tot * g * g)
            for v in range(DIM // 16):
                s_v[r, pl.ds(16 * v, 16)] = sl[v] * g
            return 0

        lax.fori_loop(0, SEGCHUNK, row, 0)
        pltpu.sync_copy(s_v, seg_sh.at[bidx_v], add=True)
        return 0

    lax.fori_loop(0, nk, body, 0)
    plsc.subcore_barrier()

    pltpu.sync_copy(seg_sh.at[pl.ds(sid * SEG_ROWS_PER_TILE, SEG_ROWS_PER_TILE)],
                    out_hbm.at[cid, pl.ds(sid * SEG_ROWS_PER_TILE, SEG_ROWS_PER_TILE)])


@jax.jit
def _segsum(agg, h, batch):
    mesh = plsc.VectorSubcoreMesh(core_axis_name="c", subcore_axis_name="s")
    return pl.kernel(
        _segsum_body,
        out_type=jax.ShapeDtypeStruct((NC, N_GRAPHS, DIM), jnp.float32),
        mesh=mesh,
        scratch_types=[
            pltpu.VMEM((SEGCHUNK, DIM), jnp.float32),
            pltpu.VMEM((SEGCHUNK, DIM), jnp.float32),
            pltpu.VMEM((SEGCHUNK, DIM), jnp.float32),
            pltpu.VMEM((SEGCHUNK,), jnp.int32),
            pltpu.VMEM((SEG_ROWS_PER_TILE, DIM), jnp.float32),
            pltpu.VMEM_SHARED((N_GRAPHS, DIM), jnp.float32),
            pltpu.SemaphoreType.DMA,
        ],
    )(agg, h, batch)


def _embed_lin_kernel(x_ref, embd_ref, w_ref, b_ref, h_ref, hx_ref):
    xb = x_ref[0, 0, :]
    iota = lax.broadcasted_iota(jnp.int32, (RB, VOCAB_PAD), 1)
    oh = (xb[:, None] == iota).astype(jnp.float32)
    h = jnp.dot(oh, embd_ref[...], preferred_element_type=jnp.float32)
    h_ref[...] = h
    hx = jnp.dot(h, w_ref[...], preferred_element_type=jnp.float32) + b_ref[...]
    hx_ref[...] = jnp.maximum(hx, 0.0)


@jax.jit
def _embed_lin(x3, embd_p, w, b):
    return pl.pallas_call(
        _embed_lin_kernel,
        grid=(NB,),
        in_specs=[
            pl.BlockSpec((1, 1, RB), lambda i: (i, 0, 0)),
            pl.BlockSpec((VOCAB_PAD, DIM), lambda i: (0, 0)),
            pl.BlockSpec((DIM, DIM), lambda i: (0, 0)),
            pl.BlockSpec((1, DIM), lambda i: (0, 0)),
        ],
        out_specs=[
            pl.BlockSpec((RB, DIM), lambda i: (i, 0)),
            pl.BlockSpec((RB, DIM), lambda i: (i, 0)),
        ],
        out_shape=[
            jax.ShapeDtypeStruct((N_NODES, DIM), jnp.float32),
            jax.ShapeDtypeStruct((N_NODES, DIM), jnp.float32),
        ],
    )(x3, embd_p, w, b)


def _layer_kernel(agg_ref, h_ref, w_ref, b_ref, hn_ref, hx_ref):
    s = agg_ref[0] + agg_ref[1] + h_ref[...]
    ss = jnp.sum(s * s, axis=1, keepdims=True)
    nrm = jnp.maximum(jnp.sqrt(ss), 1e-12)
    hn = s / nrm
    hn_ref[...] = hn
    hx = jnp.dot(hn, w_ref[...], preferred_element_type=jnp.float32) + b_ref[...]
    hx_ref[...] = jnp.maximum(hx, 0.0)


@jax.jit
def _layer(agg, h, w, b):
    return pl.pallas_call(
        _layer_kernel,
        grid=(NB,),
        in_specs=[
            pl.BlockSpec((NC, RB, DIM), lambda i: (0, i, 0)),
            pl.BlockSpec((RB, DIM), lambda i: (i, 0)),
            pl.BlockSpec((DIM, DIM), lambda i: (0, 0)),
            pl.BlockSpec((1, DIM), lambda i: (0, 0)),
        ],
        out_specs=[
            pl.BlockSpec((RB, DIM), lambda i: (i, 0)),
            pl.BlockSpec((RB, DIM), lambda i: (i, 0)),
        ],
        out_shape=[
            jax.ShapeDtypeStruct((N_NODES, DIM), jnp.float32),
            jax.ShapeDtypeStruct((N_NODES, DIM), jnp.float32),
        ],
    )(agg, h, w, b)


def _readout_kernel(seg_ref, wl_ref, bl_ref, wp_ref, bp_ref, out_ref):
    m = seg_ref[0] + seg_ref[1]
    for i in range(2):
        m = jnp.dot(m, wl_ref[i], preferred_element_type=jnp.float32)
        m = jnp.maximum(m + bl_ref[i:i + 1, :], 0.0)
    out = jnp.dot(m, wp_ref[...], preferred_element_type=jnp.float32)
    out_ref[...] = out + bp_ref[...]


@jax.jit
def _readout(seg, wl, bl, wp, bp):
    return pl.pallas_call(
        _readout_kernel,
        out_shape=jax.ShapeDtypeStruct((N_GRAPHS, 1), jnp.float32),
    )(seg, wl, bl, wp, bp)


def kernel(x, edge_index, batch, embd, W_g, b_g, W_l, b_l, W_p, b_p):
    x3 = x.astype(jnp.int32).reshape(NB, 1, RB)
    idx = edge_index.astype(jnp.int32).reshape(2, N_EDGES // ECHUNK, ECHUNK)
    batch = batch.astype(jnp.int32)
    embd_p = jnp.pad(embd, ((0, VOCAB_PAD - embd.shape[0]), (0, 0)))

    h, hx = _embed_lin(x3, embd_p, W_g[0], b_g[0].reshape(1, DIM))
    for m in range(2):
        agg = _edge_agg(hx, idx)
        h, hx = _layer(agg, h, W_g[m + 1], b_g[m + 1].reshape(1, DIM))
    agg = _edge_agg(hx, idx)

    seg = _segsum(agg, h, batch)
    props = _readout(seg, W_l, b_l, W_p, b_p.reshape(1, 1))
    return props.reshape(N_GRAPHS)


# speedup vs baseline: 12.4580x; 1.0945x over previous
import functools

import jax
import jax.numpy as jnp
from jax import lax
from jax.experimental import pallas as pl
from jax.experimental.pallas import tpu as pltpu
from jax.experimental.pallas import tpu_sc as plsc

N_NODES = 10000
N_EDGES = 320000
DIM = 128
VOCAB_PAD = 128
N_GRAPHS = 512

NC = 2
NS = 16
NW = NC * NS

EPT = N_EDGES // NW
ECHUNK = 125
ENCHUNK = EPT // ECHUNK

RCHUNK = 40
NRCHUNK = N_NODES // RCHUNK

SEGCHUNK = 80
NCHUNK_SEG = N_NODES // SEGCHUNK
SEG_ROWS_PER_TILE = N_GRAPHS // NS

RB = 1000
NB = N_NODES // RB


def _fill_zeros(zbuf_v, nrows):
    def zf(i, _):
        for j in range(DIM // 16):
            zbuf_v[i, pl.ds(j * 16, 16)] = jnp.zeros((16,), jnp.float32)
        return 0

    lax.fori_loop(0, nrows, zf, 0)


def _edge_agg_body(hx_hbm, idx_hbm, out_hbm,
                   idx0_v, idx1_v, idx2_v, idx3_v, rows0_v, rows1_v, zbuf_v,
                   agg_sh, isem0, isem1, isem2, isem3, gsem0, gsem1):
    cid = lax.axis_index("c")
    sid = lax.axis_index("s")
    tid = cid * NS + sid

    IDX = [idx0_v, idx1_v, idx2_v, idx3_v]
    ROWS = [rows0_v, rows1_v]
    ISEM = [isem0, isem1, isem2, isem3]
    GSEM = [gsem0, gsem1]

    _fill_zeros(zbuf_v, RCHUNK)
    nkr = (NRCHUNK - sid + NS - 1) // NS

    def zero_body(k, _):
        r0 = (sid + NS * k) * RCHUNK
        pltpu.sync_copy(zbuf_v, agg_sh.at[pl.ds(r0, RCHUNK)])
        return 0

    lax.fori_loop(0, nkr, zero_body, 0)
    plsc.subcore_barrier()

    cbase = tid * ENCHUNK

    def idx_load(c, q):
        pltpu.async_copy(idx_hbm.at[0, cbase + c], IDX[q].at[0], ISEM[q])
        pltpu.async_copy(idx_hbm.at[1, cbase + c], IDX[q].at[1], ISEM[q])

    def gather(c, q, r):
        pltpu.make_async_copy(idx_hbm.at[0, cbase + c], IDX[q].at[0], ISEM[q]).wait()
        pltpu.make_async_copy(idx_hbm.at[1, cbase + c], IDX[q].at[1], ISEM[q]).wait()
        pltpu.async_copy(hx_hbm.at[IDX[q].at[0]], ROWS[r], GSEM[r])

    def scatter(q, r):
        pltpu.make_async_copy(hx_hbm.at[IDX[q].at[0]], ROWS[r], GSEM[r]).wait()
        pltpu.sync_copy(ROWS[r], agg_sh.at[IDX[q].at[1]], add=True)

    idx_load(0, 0)
    idx_load(1, 1)
    idx_load(2, 2)
    gather(0, 0, 0)
    idx_load(3, 3)
    gather(1, 1, 1)
    scatter(0, 0)
    idx_load(4, 0)
    gather(2, 2, 0)
    scatter(1, 1)
    idx_load(5, 1)
    gather(3, 3, 1)
    scatter(2, 0)
    idx_load(6, 2)

    def body(j, _):
        c = 4 * j
        gather(c, 0, 0)
        scatter(3, 1)
        idx_load(c + 3, 3)
        gather(c + 1, 1, 1)
        scatter(0, 0)
        idx_load(c + 4, 0)
        gather(c + 2, 2, 0)
        scatter(1, 1)
        idx_load(c + 5, 1)
        gather(c + 3, 3, 1)
        scatter(2, 0)
        idx_load(c + 6, 2)
        return 0

    lax.fori_loop(1, ENCHUNK // 4 - 1, body, 0)

    cl = ENCHUNK - 4
    gather(cl, 0, 0)
    scatter(3, 1)
    idx_load(cl + 3, 3)
    gather(cl + 1, 1, 1)
    scatter(0, 0)
    gather(cl + 2, 2, 0)
    scatter(1, 1)
    gather(cl + 3, 3, 1)
    scatter(2, 0)
    scatter(3, 1)
    plsc.subcore_barrier()

    def wb_body(k, _):
        r0 = (sid + NS * k) * RCHUNK
        pltpu.sync_copy(agg_sh.at[pl.ds(r0, RCHUNK)],
                        out_hbm.at[cid, pl.ds(r0, RCHUNK)])
        return 0

    lax.fori_loop(0, nkr, wb_body, 0)


@jax.jit
def _edge_agg(hx, idx):
    mesh = plsc.VectorSubcoreMesh(core_axis_name="c", subcore_axis_name="s")
    return pl.kernel(
        _edge_agg_body,
        out_type=jax.ShapeDtypeStruct((NC, N_NODES, DIM), jnp.float32),
        mesh=mesh,
        scratch_types=(
            [pltpu.VMEM((2, ECHUNK), jnp.int32)] * 4
            + [pltpu.VMEM((ECHUNK, DIM), jnp.float32)] * 2
            + [pltpu.VMEM((RCHUNK, DIM), jnp.float32)]
            + [pltpu.VMEM_SHARED((N_NODES, DIM), jnp.float32)]
            + [pltpu.SemaphoreType.DMA] * 6
        ),
    )(hx, idx)


def _segsum_body(agg_hbm, h_hbm, batch_hbm, out_hbm,
                 a0_v, a1_v, s_v, bidx_v, zbuf_v, seg_sh, gsem):
    cid = lax.axis_index("c")
    sid = lax.axis_index("s")
    tid = cid * NS + sid

    _fill_zeros(zbuf_v, SEG_ROWS_PER_TILE)
    pltpu.sync_copy(zbuf_v.at[pl.ds(0, SEG_ROWS_PER_TILE)],
                    seg_sh.at[pl.ds(sid * SEG_ROWS_PER_TILE, SEG_ROWS_PER_TILE)])
    plsc.subcore_barrier()

    nk = (NCHUNK_SEG - tid + NW - 1) // NW

    def normalize_row(r):
        ss = jnp.zeros((16,), jnp.float32)
        sl = []
        for v in range(DIM // 16):
            x = (a0_v[r, pl.ds(16 * v, 16)] + a1_v[r, pl.ds(16 * v, 16)]
                 + s_v[r, pl.ds(16 * v, 16)])
            sl.append(x)
            ss = ss + x * x
        lane = lax.iota(jnp.int32, 16)
        for sh in (1, 2, 4, 8):
            ss = ss + ss.at[jnp.bitwise_xor(lane, sh)].get(
                mode="promise_in_bounds")
        tot = jnp.maximum(ss[0], jnp.float32(1e-24))
        i = lax.bitcast_convert_type(tot, jnp.int32)
        g = lax.bitcast_convert_type(
            jnp.int32(0x5F3759DF) - lax.shift_right_arithmetic(i, 1),
            jnp.float32)
        for _ in range(3):
            g = g * (1.5 - 0.5 * tot * g * g)
        for v in range(DIM // 16):
            s_v[r, pl.ds(16 * v, 16)] = sl[v] * g

    def body(k, _):
        base = (tid + NW * k) * SEGCHUNK
        pltpu.async_copy(agg_hbm.at[0, pl.ds(base, SEGCHUNK)], a0_v, gsem)
        pltpu.async_copy(agg_hbm.at[1, pl.ds(base, SEGCHUNK)], a1_v, gsem)
        pltpu.async_copy(h_hbm.at[pl.ds(base, SEGCHUNK)], s_v, gsem)
        pltpu.async_copy(batch_hbm.at[pl.ds(base, SEGCHUNK)], bidx_v, gsem)
        pltpu.make_async_copy(agg_hbm.at[0, pl.ds(base, SEGCHUNK)], a0_v, gsem).wait()
        pltpu.make_async_copy(agg_hbm.at[1, pl.ds(base, SEGCHUNK)], a1_v, gsem).wait()
        pltpu.make_async_copy(h_hbm.at[pl.ds(base, SEGCHUNK)], s_v, gsem).wait()
        pltpu.make_async_copy(batch_hbm.at[pl.ds(base, SEGCHUNK)], bidx_v, gsem).wait()

        def rows2(r, _):
            normalize_row(2 * r)
            normalize_row(2 * r + 1)
            return 0

        lax.fori_loop(0, SEGCHUNK // 2, rows2, 0)
        pltpu.sync_copy(s_v, seg_sh.at[bidx_v], add=True)
        return 0

    lax.fori_loop(0, nk, body, 0)
    plsc.subcore_barrier()

    pltpu.sync_copy(seg_sh.at[pl.ds(sid * SEG_ROWS_PER_TILE, SEG_ROWS_PER_TILE)],
                    out_hbm.at[cid, pl.ds(sid * SEG_ROWS_PER_TILE, SEG_ROWS_PER_TILE)])


@jax.jit
def _segsum(agg, h, batch):
    mesh = plsc.VectorSubcoreMesh(core_axis_name="c", subcore_axis_name="s")
    return pl.kernel(
        _segsum_body,
        out_type=jax.ShapeDtypeStruct((NC, N_GRAPHS, DIM), jnp.float32),
        mesh=mesh,
        scratch_types=[
            pltpu.VMEM((SEGCHUNK, DIM), jnp.float32),
            pltpu.VMEM((SEGCHUNK, DIM), jnp.float32),
            pltpu.VMEM((SEGCHUNK, DIM), jnp.float32),
            pltpu.VMEM((SEGCHUNK,), jnp.int32),
            pltpu.VMEM((SEG_ROWS_PER_TILE, DIM), jnp.float32),
            pltpu.VMEM_SHARED((N_GRAPHS, DIM), jnp.float32),
            pltpu.SemaphoreType.DMA,
        ],
    )(agg, h, batch)


def _embed_lin_kernel(x_ref, embd_ref, w_ref, b_ref, h_ref, hx_ref):
    xb = x_ref[0, 0, :]
    iota = lax.broadcasted_iota(jnp.int32, (RB, VOCAB_PAD), 1)
    oh = (xb[:, None] == iota).astype(jnp.float32)
    h = jnp.dot(oh, embd_ref[...], preferred_element_type=jnp.float32)
    h_ref[...] = h
    hx = jnp.dot(h, w_ref[...], preferred_element_type=jnp.float32) + b_ref[...]
    hx_ref[...] = jnp.maximum(hx, 0.0)


@jax.jit
def _embed_lin(x3, embd_p, w, b):
    return pl.pallas_call(
        _embed_lin_kernel,
        grid=(NB,),
        in_specs=[
            pl.BlockSpec((1, 1, RB), lambda i: (i, 0, 0)),
            pl.BlockSpec((VOCAB_PAD, DIM), lambda i: (0, 0)),
            pl.BlockSpec((DIM, DIM), lambda i: (0, 0)),
            pl.BlockSpec((1, DIM), lambda i: (0, 0)),
        ],
        out_specs=[
            pl.BlockSpec((RB, DIM), lambda i: (i, 0)),
            pl.BlockSpec((RB, DIM), lambda i: (i, 0)),
        ],
        out_shape=[
            jax.ShapeDtypeStruct((N_NODES, DIM), jnp.float32),
            jax.ShapeDtypeStruct((N_NODES, DIM), jnp.float32),
        ],
    )(x3, embd_p, w, b)


def _layer_kernel(agg_ref, h_ref, w_ref, b_ref, hn_ref, hx_ref):
    s = agg_ref[0] + agg_ref[1] + h_ref[...]
    ss = jnp.sum(s * s, axis=1, keepdims=True)
    nrm = jnp.maximum(jnp.sqrt(ss), 1e-12)
    hn = s / nrm
    hn_ref[...] = hn
    hx = jnp.dot(hn, w_ref[...], preferred_element_type=jnp.float32) + b_ref[...]
    hx_ref[...] = jnp.maximum(hx, 0.0)


@jax.jit
def _layer(agg, h, w, b):
    return pl.pallas_call(
        _layer_kernel,
        grid=(NB,),
        in_specs=[
            pl.BlockSpec((NC, RB, DIM), lambda i: (0, i, 0)),
            pl.BlockSpec((RB, DIM), lambda i: (i, 0)),
            pl.BlockSpec((DIM, DIM), lambda i: (0, 0)),
            pl.BlockSpec((1, DIM), lambda i: (0, 0)),
        ],
        out_specs=[
            pl.BlockSpec((RB, DIM), lambda i: (i, 0)),
            pl.BlockSpec((RB, DIM), lambda i: (i, 0)),
        ],
        out_shape=[
            jax.ShapeDtypeStruct((N_NODES, DIM), jnp.float32),
            jax.ShapeDtypeStruct((N_NODES, DIM), jnp.float32),
        ],
    )(agg, h, w, b)


def _readout_kernel(seg_ref, wl_ref, bl_ref, wp_ref, bp_ref, out_ref):
    m = seg_ref[0] + seg_ref[1]
    for i in range(2):
        m = jnp.dot(m, wl_ref[i], preferred_element_type=jnp.float32)
        m = jnp.maximum(m + bl_ref[i:i + 1, :], 0.0)
    out = jnp.dot(m, wp_ref[...], preferred_element_type=jnp.float32)
    out_ref[...] = out + bp_ref[...]


@jax.jit
def _readout(seg, wl, bl, wp, bp):
    return pl.pallas_call(
        _readout_kernel,
        out_shape=jax.ShapeDtypeStruct((N_GRAPHS, 1), jnp.float32),
    )(seg, wl, bl, wp, bp)


def kernel(x, edge_index, batch, embd, W_g, b_g, W_l, b_l, W_p, b_p):
    x3 = x.astype(jnp.int32).reshape(NB, 1, RB)
    idx = edge_index.astype(jnp.int32).reshape(2, N_EDGES // ECHUNK, ECHUNK)
    batch = batch.astype(jnp.int32)
    embd_p = jnp.pad(embd, ((0, VOCAB_PAD - embd.shape[0]), (0, 0)))

    h, hx = _embed_lin(x3, embd_p, W_g[0], b_g[0].reshape(1, DIM))
    for m in range(2):
        agg = _edge_agg(hx, idx)
        h, hx = _layer(agg, h, W_g[m + 1], b_g[m + 1].reshape(1, DIM))
    agg = _edge_agg(hx, idx)

    seg = _segsum(agg, h, batch)
    props = _readout(seg, W_l, b_l, W_p, b_p.reshape(1, 1))
    return props.reshape(N_GRAPHS)


# speedup vs baseline: 12.5265x; 1.0055x over previous
import functools

import jax
import jax.numpy as jnp
from jax import lax
from jax.experimental import pallas as pl
from jax.experimental.pallas import tpu as pltpu
from jax.experimental.pallas import tpu_sc as plsc

N_NODES = 10000
N_EDGES = 320000
DIM = 128
VOCAB_PAD = 128
N_GRAPHS = 512

NC = 2
NS = 16
NW = NC * NS

EPT = N_EDGES // NW
ECHUNK = 125
ENCHUNK = EPT // ECHUNK

RCHUNK = 40
NRCHUNK = N_NODES // RCHUNK

SEGCHUNK = 80
NCHUNK_SEG = N_NODES // SEGCHUNK
SEG_ROWS_PER_TILE = N_GRAPHS // NS

RB = 1000
NB = N_NODES // RB


def _fill_zeros(zbuf_v, nrows):
    def zf(i, _):
        for j in range(DIM // 16):
            zbuf_v[i, pl.ds(j * 16, 16)] = jnp.zeros((16,), jnp.float32)
        return 0

    lax.fori_loop(0, nrows, zf, 0)


def _edge_agg_body(hx_hbm, idx_hbm, out_hbm,
                   idx0_v, idx1_v, idx2_v, idx3_v, rows0_v, rows1_v, zbuf_v,
                   agg_sh, isem0, isem1, isem2, isem3, gsem0, gsem1):
    cid = lax.axis_index("c")
    sid = lax.axis_index("s")
    tid = cid * NS + sid

    IDX = [idx0_v, idx1_v, idx2_v, idx3_v]
    ROWS = [rows0_v, rows1_v]
    ISEM = [isem0, isem1, isem2, isem3]
    GSEM = [gsem0, gsem1]

    cbase = tid * ENCHUNK

    def idx_load(c, q):
        pltpu.async_copy(idx_hbm.at[0, cbase + c], IDX[q].at[0], ISEM[q])
        pltpu.async_copy(idx_hbm.at[1, cbase + c], IDX[q].at[1], ISEM[q])

    def gather(c, q, r):
        pltpu.make_async_copy(idx_hbm.at[0, cbase + c], IDX[q].at[0], ISEM[q]).wait()
        pltpu.make_async_copy(idx_hbm.at[1, cbase + c], IDX[q].at[1], ISEM[q]).wait()
        pltpu.async_copy(hx_hbm.at[IDX[q].at[0]], ROWS[r], GSEM[r])

    def scatter(q, r):
        pltpu.make_async_copy(hx_hbm.at[IDX[q].at[0]], ROWS[r], GSEM[r]).wait()
        pltpu.sync_copy(ROWS[r], agg_sh.at[IDX[q].at[1]], add=True)

    idx_load(0, 0)
    idx_load(1, 1)
    idx_load(2, 2)

    _fill_zeros(zbuf_v, RCHUNK)
    nkr = (NRCHUNK - sid + NS - 1) // NS

    def zero_body(k, _):
        r0 = (sid + NS * k) * RCHUNK
        pltpu.sync_copy(zbuf_v, agg_sh.at[pl.ds(r0, RCHUNK)])
        return 0

    lax.fori_loop(0, nkr, zero_body, 0)

    gather(0, 0, 0)
    idx_load(3, 3)
    gather(1, 1, 1)
    plsc.subcore_barrier()
    scatter(0, 0)
    idx_load(4, 0)
    gather(2, 2, 0)
    scatter(1, 1)
    idx_load(5, 1)
    gather(3, 3, 1)
    scatter(2, 0)
    idx_load(6, 2)

    def body(j, _):
        c = 4 * j
        gather(c, 0, 0)
        scatter(3, 1)
        idx_load(c + 3, 3)
        gather(c + 1, 1, 1)
        scatter(0, 0)
        idx_load(c + 4, 0)
        gather(c + 2, 2, 0)
        scatter(1, 1)
        idx_load(c + 5, 1)
        gather(c + 3, 3, 1)
        scatter(2, 0)
        idx_load(c + 6, 2)
        return 0

    lax.fori_loop(1, ENCHUNK // 4 - 1, body, 0)

    cl = ENCHUNK - 4
    gather(cl, 0, 0)
    scatter(3, 1)
    idx_load(cl + 3, 3)
    gather(cl + 1, 1, 1)
    scatter(0, 0)
    gather(cl + 2, 2, 0)
    scatter(1, 1)
    gather(cl + 3, 3, 1)
    scatter(2, 0)
    scatter(3, 1)
    plsc.subcore_barrier()

    def wb_body(k, _):
        r0 = (sid + NS * k) * RCHUNK
        pltpu.sync_copy(agg_sh.at[pl.ds(r0, RCHUNK)],
                        out_hbm.at[cid, pl.ds(r0, RCHUNK)])
        return 0

    lax.fori_loop(0, nkr, wb_body, 0)


@jax.jit
def _edge_agg(hx, idx):
    mesh = plsc.VectorSubcoreMesh(core_axis_name="c", subcore_axis_name="s")
    return pl.kernel(
        _edge_agg_body,
        out_type=jax.ShapeDtypeStruct((NC, N_NODES, DIM), jnp.float32),
        mesh=mesh,
        scratch_types=(
            [pltpu.VMEM((2, ECHUNK), jnp.int32)] * 4
            + [pltpu.VMEM((ECHUNK, DIM), jnp.float32)] * 2
            + [pltpu.VMEM((RCHUNK, DIM), jnp.float32)]
            + [pltpu.VMEM_SHARED((N_NODES, DIM), jnp.float32)]
            + [pltpu.SemaphoreType.DMA] * 6
        ),
    )(hx, idx)


def _segsum_body(agg_hbm, h_hbm, batch_hbm, out_hbm,
                 a0_0, a1_0, s_0, b_0, a0_1, a1_1, s_1, b_1,
                 zbuf_v, seg_sh, gsem0, gsem1):
    cid = lax.axis_index("c")
    sid = lax.axis_index("s")
    tid = cid * NS + sid

    A0 = [a0_0, a0_1]
    A1 = [a1_0, a1_1]
    S = [s_0, s_1]
    B = [b_0, b_1]
    SEM = [gsem0, gsem1]

    nk = (NCHUNK_SEG - tid + NW - 1) // NW

    def load(k, b):
        base = (tid + NW * k) * SEGCHUNK
        pltpu.async_copy(agg_hbm.at[0, pl.ds(base, SEGCHUNK)], A0[b], SEM[b])
        pltpu.async_copy(agg_hbm.at[1, pl.ds(base, SEGCHUNK)], A1[b], SEM[b])
        pltpu.async_copy(h_hbm.at[pl.ds(base, SEGCHUNK)], S[b], SEM[b])
        pltpu.async_copy(batch_hbm.at[pl.ds(base, SEGCHUNK)], B[b], SEM[b])

    def wait_load(k, b):
        base = (tid + NW * k) * SEGCHUNK
        pltpu.make_async_copy(agg_hbm.at[0, pl.ds(base, SEGCHUNK)], A0[b], SEM[b]).wait()
        pltpu.make_async_copy(agg_hbm.at[1, pl.ds(base, SEGCHUNK)], A1[b], SEM[b]).wait()
        pltpu.make_async_copy(h_hbm.at[pl.ds(base, SEGCHUNK)], S[b], SEM[b]).wait()
        pltpu.make_async_copy(batch_hbm.at[pl.ds(base, SEGCHUNK)], B[b], SEM[b]).wait()

    def normalize_row(b, r):
        ss = jnp.zeros((16,), jnp.float32)
        sl = []
        for v in range(DIM // 16):
            x = (A0[b][r, pl.ds(16 * v, 16)] + A1[b][r, pl.ds(16 * v, 16)]
                 + S[b][r, pl.ds(16 * v, 16)])
            sl.append(x)
            ss = ss + x * x
        lane = lax.iota(jnp.int32, 16)
        for sh in (1, 2, 4, 8):
            ss = ss + ss.at[jnp.bitwise_xor(lane, sh)].get(
                mode="promise_in_bounds")
        tot = jnp.maximum(ss[0], jnp.float32(1e-24))
        i = lax.bitcast_convert_type(tot, jnp.int32)
        g = lax.bitcast_convert_type(
            jnp.int32(0x5F3759DF) - lax.shift_right_arithmetic(i, 1),
            jnp.float32)
        for _ in range(3):
            g = g * (1.5 - 0.5 * tot * g * g)
        for v in range(DIM // 16):
            S[b][r, pl.ds(16 * v, 16)] = sl[v] * g

    load(0, 0)
    load(1, 1)

    _fill_zeros(zbuf_v, SEG_ROWS_PER_TILE)
    pltpu.sync_copy(zbuf_v.at[pl.ds(0, SEG_ROWS_PER_TILE)],
                    seg_sh.at[pl.ds(sid * SEG_ROWS_PER_TILE, SEG_ROWS_PER_TILE)])
    plsc.subcore_barrier()

    for k in range(4):
        b = k % 2

        @pl.when(k < nk)
        def _():
            wait_load(k, b)

            def rows2(r, _):
                normalize_row(b, 2 * r)
                normalize_row(b, 2 * r + 1)
                return 0

            lax.fori_loop(0, SEGCHUNK // 2, rows2, 0)
            pltpu.sync_copy(S[b], seg_sh.at[B[b]], add=True)

            @pl.when(k + 2 < nk)
            def _():
                load(k + 2, b)

    plsc.subcore_barrier()

    pltpu.sync_copy(seg_sh.at[pl.ds(sid * SEG_ROWS_PER_TILE, SEG_ROWS_PER_TILE)],
                    out_hbm.at[cid, pl.ds(sid * SEG_ROWS_PER_TILE, SEG_ROWS_PER_TILE)])


@jax.jit
def _segsum(agg, h, batch):
    mesh = plsc.VectorSubcoreMesh(core_axis_name="c", subcore_axis_name="s")
    return pl.kernel(
        _segsum_body,
        out_type=jax.ShapeDtypeStruct((NC, N_GRAPHS, DIM), jnp.float32),
        mesh=mesh,
        scratch_types=(
            ([pltpu.VMEM((SEGCHUNK, DIM), jnp.float32)] * 3
             + [pltpu.VMEM((SEGCHUNK,), jnp.int32)]) * 2
            + [pltpu.VMEM((SEG_ROWS_PER_TILE, DIM), jnp.float32)]
            + [pltpu.VMEM_SHARED((N_GRAPHS, DIM), jnp.float32)]
            + [pltpu.SemaphoreType.DMA] * 2
        ),
    )(agg, h, batch)


def _embed_lin_kernel(x_ref, embd_ref, w_ref, b_ref, h_ref, hx_ref):
    xb = x_ref[0, 0, :]
    iota = lax.broadcasted_iota(jnp.int32, (RB, VOCAB_PAD), 1)
    oh = (xb[:, None] == iota).astype(jnp.float32)
    h = jnp.dot(oh, embd_ref[...], preferred_element_type=jnp.float32)
    h_ref[...] = h
    hx = jnp.dot(h, w_ref[...], preferred_element_type=jnp.float32) + b_ref[...]
    hx_ref[...] = jnp.maximum(hx, 0.0)


@jax.jit
def _embed_lin(x3, embd_p, w, b):
    return pl.pallas_call(
        _embed_lin_kernel,
        grid=(NB,),
        in_specs=[
            pl.BlockSpec((1, 1, RB), lambda i: (i, 0, 0)),
            pl.BlockSpec((VOCAB_PAD, DIM), lambda i: (0, 0)),
            pl.BlockSpec((DIM, DIM), lambda i: (0, 0)),
            pl.BlockSpec((1, DIM), lambda i: (0, 0)),
        ],
        out_specs=[
            pl.BlockSpec((RB, DIM), lambda i: (i, 0)),
            pl.BlockSpec((RB, DIM), lambda i: (i, 0)),
        ],
        out_shape=[
            jax.ShapeDtypeStruct((N_NODES, DIM), jnp.float32),
            jax.ShapeDtypeStruct((N_NODES, DIM), jnp.float32),
        ],
    )(x3, embd_p, w, b)


def _layer_kernel(agg_ref, h_ref, w_ref, b_ref, hn_ref, hx_ref):
    s = agg_ref[0] + agg_ref[1] + h_ref[...]
    ss = jnp.sum(s * s, axis=1, keepdims=True)
    nrm = jnp.maximum(jnp.sqrt(ss), 1e-12)
    hn = s / nrm
    hn_ref[...] = hn
    hx = jnp.dot(hn, w_ref[...], preferred_element_type=jnp.float32) + b_ref[...]
    hx_ref[...] = jnp.maximum(hx, 0.0)


@jax.jit
def _layer(agg, h, w, b):
    return pl.pallas_call(
        _layer_kernel,
        grid=(NB,),
        in_specs=[
            pl.BlockSpec((NC, RB, DIM), lambda i: (0, i, 0)),
            pl.BlockSpec((RB, DIM), lambda i: (i, 0)),
            pl.BlockSpec((DIM, DIM), lambda i: (0, 0)),
            pl.BlockSpec((1, DIM), lambda i: (0, 0)),
        ],
        out_specs=[
            pl.BlockSpec((RB, DIM), lambda i: (i, 0)),
            pl.BlockSpec((RB, DIM), lambda i: (i, 0)),
        ],
        out_shape=[
            jax.ShapeDtypeStruct((N_NODES, DIM), jnp.float32),
            jax.ShapeDtypeStruct((N_NODES, DIM), jnp.float32),
        ],
    )(agg, h, w, b)


def _readout_kernel(seg_ref, wl_ref, bl_ref, wp_ref, bp_ref, out_ref):
    m = seg_ref[0] + seg_ref[1]
    for i in range(2):
        m = jnp.dot(m, wl_ref[i], preferred_element_type=jnp.float32)
        m = jnp.maximum(m + bl_ref[i:i + 1, :], 0.0)
    out = jnp.dot(m, wp_ref[...], preferred_element_type=jnp.float32)
    out_ref[...] = out + bp_ref[...]


@jax.jit
def _readout(seg, wl, bl, wp, bp):
    return pl.pallas_call(
        _readout_kernel,
        out_shape=jax.ShapeDtypeStruct((N_GRAPHS, 1), jnp.float32),
    )(seg, wl, bl, wp, bp)


def kernel(x, edge_index, batch, embd, W_g, b_g, W_l, b_l, W_p, b_p):
    x3 = x.astype(jnp.int32).reshape(NB, 1, RB)
    idx = edge_index.astype(jnp.int32).reshape(2, N_EDGES // ECHUNK, ECHUNK)
    batch = batch.astype(jnp.int32)
    embd_p = jnp.pad(embd, ((0, VOCAB_PAD - embd.shape[0]), (0, 0)))

    h, hx = _embed_lin(x3, embd_p, W_g[0], b_g[0].reshape(1, DIM))
    for m in range(2):
        agg = _edge_agg(hx, idx)
        h, hx = _layer(agg, h, W_g[m + 1], b_g[m + 1].reshape(1, DIM))
    agg = _edge_agg(hx, idx)

    seg = _segsum(agg, h, batch)
    props = _readout(seg, W_l, b_l, W_p, b_p.reshape(1, 1))
    return props.reshape(N_GRAPHS)


# speedup vs baseline: 12.5990x; 1.0058x over previous
import functools

import jax
import jax.numpy as jnp
from jax import lax
from jax.experimental import pallas as pl
from jax.experimental.pallas import tpu as pltpu
from jax.experimental.pallas import tpu_sc as plsc

N_NODES = 10000
N_EDGES = 320000
DIM = 128
VOCAB_PAD = 128
N_GRAPHS = 512

NC = 2
NS = 16
NW = NC * NS

ECHUNK = 128
NCH = N_EDGES // ECHUNK

RCHUNK = 40
NRCHUNK = N_NODES // RCHUNK

SEGCHUNK = 80
NCHUNK_SEG = N_NODES // SEGCHUNK
SEG_ROWS_PER_TILE = N_GRAPHS // NS

RB = 1000
NB = N_NODES // RB


def _fill_zeros(zbuf_v, nrows):
    def zf(i, _):
        for j in range(DIM // 16):
            zbuf_v[i, pl.ds(j * 16, 16)] = jnp.zeros((16,), jnp.float32)
        return 0

    lax.fori_loop(0, nrows, zf, 0)


def _edge_agg_body(hx_hbm, idx_hbm, out_hbm,
                   idx0_v, idx1_v, idx2_v, idx3_v, rows0_v, rows1_v, zbuf_v,
                   agg_sh, isem0, isem1, isem2, isem3, gsem0, gsem1):
    cid = lax.axis_index("c")
    sid = lax.axis_index("s")
    tid = cid * NS + sid

    IDX = [idx0_v, idx1_v, idx2_v, idx3_v]
    ROWS = [rows0_v, rows1_v]
    ISEM = [isem0, isem1, isem2, isem3]
    GSEM = [gsem0, gsem1]

    cbase = 78 * tid + jnp.minimum(tid, 4)
    extra = tid < 4

    def idx_load(k, q):
        pltpu.async_copy(idx_hbm.at[0, cbase + k], IDX[q].at[0], ISEM[q])
        pltpu.async_copy(idx_hbm.at[1, cbase + k], IDX[q].at[1], ISEM[q])

    def gather(k, q, r):
        pltpu.make_async_copy(idx_hbm.at[0, cbase + k], IDX[q].at[0], ISEM[q]).wait()
        pltpu.make_async_copy(idx_hbm.at[1, cbase + k], IDX[q].at[1], ISEM[q]).wait()
        pltpu.async_copy(hx_hbm.at[IDX[q].at[0]], ROWS[r], GSEM[r])

    def scatter(q, r):
        pltpu.make_async_copy(hx_hbm.at[IDX[q].at[0]], ROWS[r], GSEM[r]).wait()
        pltpu.sync_copy(ROWS[r], agg_sh.at[IDX[q].at[1]], add=True)

    idx_load(0, 0)
    idx_load(1, 1)
    idx_load(2, 2)

    _fill_zeros(zbuf_v, RCHUNK)
    nkr = (NRCHUNK - sid + NS - 1) // NS

    def zero_body(k, _):
        r0 = (sid + NS * k) * RCHUNK
        pltpu.sync_copy(zbuf_v, agg_sh.at[pl.ds(r0, RCHUNK)])
        return 0

    lax.fori_loop(0, nkr, zero_body, 0)

    gather(0, 0, 0)
    idx_load(3, 3)
    gather(1, 1, 1)
    plsc.subcore_barrier()
    scatter(0, 0)
    idx_load(4, 0)
    gather(2, 2, 0)
    scatter(1, 1)
    idx_load(5, 1)
    gather(3, 3, 1)
    scatter(2, 0)
    idx_load(6, 2)

    def body(j, _):
        k = 4 * j
        gather(k, 0, 0)
        scatter(3, 1)
        idx_load(k + 3, 3)
        gather(k + 1, 1, 1)
        scatter(0, 0)
        idx_load(k + 4, 0)
        gather(k + 2, 2, 0)
        scatter(1, 1)
        idx_load(k + 5, 1)
        gather(k + 3, 3, 1)
        scatter(2, 0)
        idx_load(k + 6, 2)
        return 0

    lax.fori_loop(1, 18, body, 0)

    gather(72, 0, 0)
    scatter(3, 1)
    idx_load(75, 3)
    gather(73, 1, 1)
    scatter(0, 0)
    idx_load(76, 0)
    gather(74, 2, 0)
    scatter(1, 1)
    idx_load(77, 1)
    gather(75, 3, 1)
    scatter(2, 0)

    @pl.when(extra)
    def _():
        idx_load(78, 2)

    gather(76, 0, 0)
    scatter(3, 1)
    gather(77, 1, 1)
    scatter(0, 0)

    @pl.when(extra)
    def _():
        gather(78, 2, 0)

    scatter(1, 1)

    @pl.when(extra)
    def _():
        scatter(2, 0)

    plsc.subcore_barrier()

    def wb_body(k, _):
        r0 = (sid + NS * k) * RCHUNK
        pltpu.sync_copy(agg_sh.at[pl.ds(r0, RCHUNK)],
                        out_hbm.at[cid, pl.ds(r0, RCHUNK)])
        return 0

    lax.fori_loop(0, nkr, wb_body, 0)


@jax.jit
def _edge_agg(hx, idx):
    mesh = plsc.VectorSubcoreMesh(core_axis_name="c", subcore_axis_name="s")
    return pl.kernel(
        _edge_agg_body,
        out_type=jax.ShapeDtypeStruct((NC, N_NODES, DIM), jnp.float32),
        mesh=mesh,
        scratch_types=(
            [pltpu.VMEM((2, ECHUNK), jnp.int32)] * 4
            + [pltpu.VMEM((ECHUNK, DIM), jnp.float32)] * 2
            + [pltpu.VMEM((RCHUNK, DIM), jnp.float32)]
            + [pltpu.VMEM_SHARED((N_NODES, DIM), jnp.float32)]
            + [pltpu.SemaphoreType.DMA] * 6
        ),
    )(hx, idx)


def _segsum_body(agg_hbm, h_hbm, batch_hbm, out_hbm,
                 a0_0, a1_0, s_0, b_0, a0_1, a1_1, s_1, b_1,
                 zbuf_v, seg_sh, gsem0, gsem1):
    cid = lax.axis_index("c")
    sid = lax.axis_index("s")
    tid = cid * NS + sid

    A0 = [a0_0, a0_1]
    A1 = [a1_0, a1_1]
    S = [s_0, s_1]
    B = [b_0, b_1]
    SEM = [gsem0, gsem1]

    nk = (NCHUNK_SEG - tid + NW - 1) // NW

    def load(k, b):
        base = (tid + NW * k) * SEGCHUNK
        pltpu.async_copy(agg_hbm.at[0, pl.ds(base, SEGCHUNK)], A0[b], SEM[b])
        pltpu.async_copy(agg_hbm.at[1, pl.ds(base, SEGCHUNK)], A1[b], SEM[b])
        pltpu.async_copy(h_hbm.at[pl.ds(base, SEGCHUNK)], S[b], SEM[b])
        pltpu.async_copy(batch_hbm.at[pl.ds(base, SEGCHUNK)], B[b], SEM[b])

    def wait_load(k, b):
        base = (tid + NW * k) * SEGCHUNK
        pltpu.make_async_copy(agg_hbm.at[0, pl.ds(base, SEGCHUNK)], A0[b], SEM[b]).wait()
        pltpu.make_async_copy(agg_hbm.at[1, pl.ds(base, SEGCHUNK)], A1[b], SEM[b]).wait()
        pltpu.make_async_copy(h_hbm.at[pl.ds(base, SEGCHUNK)], S[b], SEM[b]).wait()
        pltpu.make_async_copy(batch_hbm.at[pl.ds(base, SEGCHUNK)], B[b], SEM[b]).wait()

    def normalize_row(b, r):
        ss = jnp.zeros((16,), jnp.float32)
        sl = []
        for v in range(DIM // 16):
            x = (A0[b][r, pl.ds(16 * v, 16)] + A1[b][r, pl.ds(16 * v, 16)]
                 + S[b][r, pl.ds(16 * v, 16)])
            sl.append(x)
            ss = ss + x * x
        lane = lax.iota(jnp.int32, 16)
        for sh in (1, 2, 4, 8):
            ss = ss + ss.at[jnp.bitwise_xor(lane, sh)].get(
                mode="promise_in_bounds")
        tot = jnp.maximum(ss[0], jnp.float32(1e-24))
        i = lax.bitcast_convert_type(tot, jnp.int32)
        g = lax.bitcast_convert_type(
            jnp.int32(0x5F3759DF) - lax.shift_right_arithmetic(i, 1),
            jnp.float32)
        for _ in range(3):
            g = g * (1.5 - 0.5 * tot * g * g)
        for v in range(DIM // 16):
            S[b][r, pl.ds(16 * v, 16)] = sl[v] * g

    load(0, 0)
    load(1, 1)

    _fill_zeros(zbuf_v, SEG_ROWS_PER_TILE)
    pltpu.sync_copy(zbuf_v.at[pl.ds(0, SEG_ROWS_PER_TILE)],
                    seg_sh.at[pl.ds(sid * SEG_ROWS_PER_TILE, SEG_ROWS_PER_TILE)])
    plsc.subcore_barrier()

    for k in range(4):
        b = k % 2

        @pl.when(k < nk)
        def _():
            wait_load(k, b)

            def rows2(r, _):
                normalize_row(b, 2 * r)
                normalize_row(b, 2 * r + 1)
                return 0

            lax.fori_loop(0, SEGCHUNK // 2, rows2, 0)
            pltpu.sync_copy(S[b], seg_sh.at[B[b]], add=True)

            @pl.when(k + 2 < nk)
            def _():
                load(k + 2, b)

    plsc.subcore_barrier()

    pltpu.sync_copy(seg_sh.at[pl.ds(sid * SEG_ROWS_PER_TILE, SEG_ROWS_PER_TILE)],
                    out_hbm.at[cid, pl.ds(sid * SEG_ROWS_PER_TILE, SEG_ROWS_PER_TILE)])


@jax.jit
def _segsum(agg, h, batch):
    mesh = plsc.VectorSubcoreMesh(core_axis_name="c", subcore_axis_name="s")
    return pl.kernel(
        _segsum_body,
        out_type=jax.ShapeDtypeStruct((NC, N_GRAPHS, DIM), jnp.float32),
        mesh=mesh,
        scratch_types=(
            ([pltpu.VMEM((SEGCHUNK, DIM), jnp.float32)] * 3
             + [pltpu.VMEM((SEGCHUNK,), jnp.int32)]) * 2
            + [pltpu.VMEM((SEG_ROWS_PER_TILE, DIM), jnp.float32)]
            + [pltpu.VMEM_SHARED((N_GRAPHS, DIM), jnp.float32)]
            + [pltpu.SemaphoreType.DMA] * 2
        ),
    )(agg, h, batch)


def _embed_lin_kernel(x_ref, embd_ref, w_ref, b_ref, h_ref, hx_ref):
    xb = x_ref[0, 0, :]
    iota = lax.broadcasted_iota(jnp.int32, (RB, VOCAB_PAD), 1)
    oh = (xb[:, None] == iota).astype(jnp.float32)
    h = jnp.dot(oh, embd_ref[...], preferred_element_type=jnp.float32)
    h_ref[...] = h
    hx = jnp.dot(h, w_ref[...], preferred_element_type=jnp.float32) + b_ref[...]
    hx_ref[...] = jnp.maximum(hx, 0.0)


@jax.jit
def _embed_lin(x3, embd_p, w, b):
    return pl.pallas_call(
        _embed_lin_kernel,
        grid=(NB,),
        in_specs=[
            pl.BlockSpec((1, 1, RB), lambda i: (i, 0, 0)),
            pl.BlockSpec((VOCAB_PAD, DIM), lambda i: (0, 0)),
            pl.BlockSpec((DIM, DIM), lambda i: (0, 0)),
            pl.BlockSpec((1, DIM), lambda i: (0, 0)),
        ],
        out_specs=[
            pl.BlockSpec((RB, DIM), lambda i: (i, 0)),
            pl.BlockSpec((RB, DIM), lambda i: (i, 0)),
        ],
        out_shape=[
            jax.ShapeDtypeStruct((N_NODES, DIM), jnp.float32),
            jax.ShapeDtypeStruct((N_NODES, DIM), jnp.float32),
        ],
    )(x3, embd_p, w, b)


def _layer_kernel(agg_ref, h_ref, w_ref, b_ref, hn_ref, hx_ref):
    s = agg_ref[0] + agg_ref[1] + h_ref[...]
    ss = jnp.sum(s * s, axis=1, keepdims=True)
    nrm = jnp.maximum(jnp.sqrt(ss), 1e-12)
    hn = s / nrm
    hn_ref[...] = hn
    hx = jnp.dot(hn, w_ref[...], preferred_element_type=jnp.float32) + b_ref[...]
    hx_ref[...] = jnp.maximum(hx, 0.0)


@jax.jit
def _layer(agg, h, w, b):
    return pl.pallas_call(
        _layer_kernel,
        grid=(NB,),
        in_specs=[
            pl.BlockSpec((NC, RB, DIM), lambda i: (0, i, 0)),
            pl.BlockSpec((RB, DIM), lambda i: (i, 0)),
            pl.BlockSpec((DIM, DIM), lambda i: (0, 0)),
            pl.BlockSpec((1, DIM), lambda i: (0, 0)),
        ],
        out_specs=[
            pl.BlockSpec((RB, DIM), lambda i: (i, 0)),
            pl.BlockSpec((RB, DIM), lambda i: (i, 0)),
        ],
        out_shape=[
            jax.ShapeDtypeStruct((N_NODES, DIM), jnp.float32),
            jax.ShapeDtypeStruct((N_NODES, DIM), jnp.float32),
        ],
    )(agg, h, w, b)


def _readout_kernel(seg_ref, wl_ref, bl_ref, wp_ref, bp_ref, out_ref):
    m = seg_ref[0] + seg_ref[1]
    for i in range(2):
        m = jnp.dot(m, wl_ref[i], preferred_element_type=jnp.float32)
        m = jnp.maximum(m + bl_ref[i:i + 1, :], 0.0)
    out = jnp.dot(m, wp_ref[...], preferred_element_type=jnp.float32)
    out_ref[...] = out + bp_ref[...]


@jax.jit
def _readout(seg, wl, bl, wp, bp):
    return pl.pallas_call(
        _readout_kernel,
        out_shape=jax.ShapeDtypeStruct((N_GRAPHS, 1), jnp.float32),
    )(seg, wl, bl, wp, bp)


def kernel(x, edge_index, batch, embd, W_g, b_g, W_l, b_l, W_p, b_p):
    x3 = x.astype(jnp.int32).reshape(NB, 1, RB)
    idx = edge_index.astype(jnp.int32).reshape(2, NCH, ECHUNK)
    batch = batch.astype(jnp.int32)
    embd_p = jnp.pad(embd, ((0, VOCAB_PAD - embd.shape[0]), (0, 0)))

    h, hx = _embed_lin(x3, embd_p, W_g[0], b_g[0].reshape(1, DIM))
    for m in range(2):
        agg = _edge_agg(hx, idx)
        h, hx = _layer(agg, h, W_g[m + 1], b_g[m + 1].reshape(1, DIM))
    agg = _edge_agg(hx, idx)

    seg = _segsum(agg, h, batch)
    props = _readout(seg, W_l, b_l, W_p, b_p.reshape(1, 1))
    return props.reshape(N_GRAPHS)


# speedup vs baseline: 12.6184x; 1.0015x over previous
import functools

import jax
import jax.numpy as jnp
from jax import lax
from jax.experimental import pallas as pl
from jax.experimental.pallas import tpu as pltpu
from jax.experimental.pallas import tpu_sc as plsc

N_NODES = 10000
N_EDGES = 320000
DIM = 128
VOCAB_PAD = 128
N_GRAPHS = 512

NC = 2
NS = 16
NW = NC * NS

ECHUNK = 128
NCH = N_EDGES // ECHUNK

RCHUNK = 40
NRCHUNK = N_NODES // RCHUNK

SEGCHUNK = 80
NCHUNK_SEG = N_NODES // SEGCHUNK
SEG_ROWS_PER_TILE = N_GRAPHS // NS

RB = 1000
NB = N_NODES // RB


def _fill_zeros(zbuf_v, nrows):
    def zf(i, _):
        for j in range(DIM // 16):
            zbuf_v[i, pl.ds(j * 16, 16)] = jnp.zeros((16,), jnp.float32)
        return 0

    lax.fori_loop(0, nrows, zf, 0)


def _edge_agg_body(hx_hbm, idx_hbm, out_hbm,
                   idx0_v, idx1_v, idx2_v, idx3_v, rows0_v, rows1_v, zbuf_v,
                   agg_sh, isem0, isem1, isem2, isem3, gsem0, gsem1):
    cid = lax.axis_index("c")
    sid = lax.axis_index("s")
    tid = cid * NS + sid

    IDX = [idx0_v, idx1_v, idx2_v, idx3_v]
    ROWS = [rows0_v, rows1_v]
    ISEM = [isem0, isem1, isem2, isem3]
    GSEM = [gsem0, gsem1]

    cbase = 78 * tid + jnp.minimum(tid, 4)
    extra = tid < 4

    def idx_load(k, q):
        pltpu.async_copy(idx_hbm.at[0, cbase + k], IDX[q].at[0], ISEM[q])
        pltpu.async_copy(idx_hbm.at[1, cbase + k], IDX[q].at[1], ISEM[q])

    def gather(k, q, r):
        pltpu.make_async_copy(idx_hbm.at[0, cbase + k], IDX[q].at[0], ISEM[q]).wait()
        pltpu.make_async_copy(idx_hbm.at[1, cbase + k], IDX[q].at[1], ISEM[q]).wait()
        pltpu.async_copy(hx_hbm.at[IDX[q].at[0]], ROWS[r], GSEM[r])

    def scatter(q, r):
        pltpu.make_async_copy(hx_hbm.at[IDX[q].at[0]], ROWS[r], GSEM[r]).wait()
        pltpu.sync_copy(ROWS[r], agg_sh.at[IDX[q].at[1]], add=True)

    idx_load(0, 0)
    idx_load(1, 1)
    idx_load(2, 2)

    _fill_zeros(zbuf_v, RCHUNK)
    nkr = (NRCHUNK - sid + NS - 1) // NS

    def zero_body(k, _):
        r0 = (sid + NS * k) * RCHUNK
        pltpu.sync_copy(zbuf_v, agg_sh.at[pl.ds(r0, RCHUNK)])
        return 0

    lax.fori_loop(0, nkr, zero_body, 0)

    gather(0, 0, 0)
    idx_load(3, 3)
    gather(1, 1, 1)
    plsc.subcore_barrier()
    scatter(0, 0)
    idx_load(4, 0)
    gather(2, 2, 0)
    scatter(1, 1)
    idx_load(5, 1)
    gather(3, 3, 1)
    scatter(2, 0)
    idx_load(6, 2)

    def body(j, _):
        k = 4 * j
        gather(k, 0, 0)
        scatter(3, 1)
        idx_load(k + 3, 3)
        gather(k + 1, 1, 1)
        scatter(0, 0)
        idx_load(k + 4, 0)
        gather(k + 2, 2, 0)
        scatter(1, 1)
        idx_load(k + 5, 1)
        gather(k + 3, 3, 1)
        scatter(2, 0)
        idx_load(k + 6, 2)
        return 0

    lax.fori_loop(1, 18, body, 0)

    gather(72, 0, 0)
    scatter(3, 1)
    idx_load(75, 3)
    gather(73, 1, 1)
    scatter(0, 0)
    idx_load(76, 0)
    gather(74, 2, 0)
    scatter(1, 1)
    idx_load(77, 1)
    gather(75, 3, 1)
    scatter(2, 0)

    @pl.when(extra)
    def _():
        idx_load(78, 2)

    gather(76, 0, 0)
    scatter(3, 1)
    gather(77, 1, 1)
    scatter(0, 0)

    @pl.when(extra)
    def _():
        gather(78, 2, 0)

    scatter(1, 1)

    @pl.when(extra)
    def _():
        scatter(2, 0)

    plsc.subcore_barrier()

    def wb_body(k, _):
        r0 = (sid + NS * k) * RCHUNK
        pltpu.sync_copy(agg_sh.at[pl.ds(r0, RCHUNK)],
                        out_hbm.at[cid, pl.ds(r0, RCHUNK)])
        return 0

    lax.fori_loop(0, nkr, wb_body, 0)


@jax.jit
def _edge_agg(hx, idx):
    mesh = plsc.VectorSubcoreMesh(core_axis_name="c", subcore_axis_name="s")
    return pl.kernel(
        _edge_agg_body,
        out_type=jax.ShapeDtypeStruct((NC, N_NODES, DIM), jnp.float32),
        mesh=mesh,
        scratch_types=(
            [pltpu.VMEM((2, ECHUNK), jnp.int32)] * 4
            + [pltpu.VMEM((ECHUNK, DIM), jnp.float32)] * 2
            + [pltpu.VMEM((RCHUNK, DIM), jnp.float32)]
            + [pltpu.VMEM_SHARED((N_NODES, DIM), jnp.float32)]
            + [pltpu.SemaphoreType.DMA] * 6
        ),
    )(hx, idx)


def _segsum_body(agg_hbm, h_hbm, batch_hbm, out_hbm,
                 a0_0, a1_0, s_0, b_0, a0_1, a1_1, s_1, b_1,
                 zbuf_v, seg_sh, gsem0, gsem1):
    cid = lax.axis_index("c")
    sid = lax.axis_index("s")
    tid = cid * NS + sid

    A0 = [a0_0, a0_1]
    A1 = [a1_0, a1_1]
    S = [s_0, s_1]
    B = [b_0, b_1]
    SEM = [gsem0, gsem1]

    nk = (NCHUNK_SEG - tid + NW - 1) // NW

    def load(k, b):
        base = (tid + NW * k) * SEGCHUNK
        pltpu.async_copy(agg_hbm.at[0, pl.ds(base, SEGCHUNK)], A0[b], SEM[b])
        pltpu.async_copy(agg_hbm.at[1, pl.ds(base, SEGCHUNK)], A1[b], SEM[b])
        pltpu.async_copy(h_hbm.at[pl.ds(base, SEGCHUNK)], S[b], SEM[b])
        pltpu.async_copy(batch_hbm.at[pl.ds(base, SEGCHUNK)], B[b], SEM[b])

    def wait_load(k, b):
        base = (tid + NW * k) * SEGCHUNK
        pltpu.make_async_copy(agg_hbm.at[0, pl.ds(base, SEGCHUNK)], A0[b], SEM[b]).wait()
        pltpu.make_async_copy(agg_hbm.at[1, pl.ds(base, SEGCHUNK)], A1[b], SEM[b]).wait()
        pltpu.make_async_copy(h_hbm.at[pl.ds(base, SEGCHUNK)], S[b], SEM[b]).wait()
        pltpu.make_async_copy(batch_hbm.at[pl.ds(base, SEGCHUNK)], B[b], SEM[b]).wait()

    def normalize_row(b, r):
        ss = jnp.zeros((16,), jnp.float32)
        sl = []
        for v in range(DIM // 16):
            x = (A0[b][r, pl.ds(16 * v, 16)] + A1[b][r, pl.ds(16 * v, 16)]
                 + S[b][r, pl.ds(16 * v, 16)])
            sl.append(x)
            ss = ss + x * x
        lane = lax.iota(jnp.int32, 16)
        for sh in (1, 2, 4, 8):
            ss = ss + ss.at[jnp.bitwise_xor(lane, sh)].get(
                mode="promise_in_bounds")
        tot = jnp.maximum(ss[0], jnp.float32(1e-24))
        i = lax.bitcast_convert_type(tot, jnp.int32)
        g = lax.bitcast_convert_type(
            jnp.int32(0x5F3759DF) - lax.shift_right_arithmetic(i, 1),
            jnp.float32)
        for _ in range(3):
            g = g * (1.5 - 0.5 * tot * g * g)
        for v in range(DIM // 16):
            S[b][r, pl.ds(16 * v, 16)] = sl[v] * g

    load(0, 0)
    load(1, 1)

    _fill_zeros(zbuf_v, SEG_ROWS_PER_TILE)
    pltpu.sync_copy(zbuf_v.at[pl.ds(0, SEG_ROWS_PER_TILE)],
                    seg_sh.at[pl.ds(sid * SEG_ROWS_PER_TILE, SEG_ROWS_PER_TILE)])
    plsc.subcore_barrier()

    for k in range(4):
        b = k % 2

        @pl.when(k < nk)
        def _():
            wait_load(k, b)

            def rows4(r, _):
                for u in range(4):
                    normalize_row(b, 4 * r + u)
                return 0

            lax.fori_loop(0, SEGCHUNK // 4, rows4, 0)
            pltpu.sync_copy(S[b], seg_sh.at[B[b]], add=True)

            @pl.when(k + 2 < nk)
            def _():
                load(k + 2, b)

    plsc.subcore_barrier()

    pltpu.sync_copy(seg_sh.at[pl.ds(sid * SEG_ROWS_PER_TILE, SEG_ROWS_PER_TILE)],
                    out_hbm.at[cid, pl.ds(sid * SEG_ROWS_PER_TILE, SEG_ROWS_PER_TILE)])


@jax.jit
def _segsum(agg, h, batch):
    mesh = plsc.VectorSubcoreMesh(core_axis_name="c", subcore_axis_name="s")
    return pl.kernel(
        _segsum_body,
        out_type=jax.ShapeDtypeStruct((NC, N_GRAPHS, DIM), jnp.float32),
        mesh=mesh,
        scratch_types=(
            ([pltpu.VMEM((SEGCHUNK, DIM), jnp.float32)] * 3
             + [pltpu.VMEM((SEGCHUNK,), jnp.int32)]) * 2
            + [pltpu.VMEM((SEG_ROWS_PER_TILE, DIM), jnp.float32)]
            + [pltpu.VMEM_SHARED((N_GRAPHS, DIM), jnp.float32)]
            + [pltpu.SemaphoreType.DMA] * 2
        ),
    )(agg, h, batch)


def _embed_lin_kernel(x_ref, embd_ref, w_ref, b_ref, h_ref, hx_ref):
    xb = x_ref[0, 0, :]
    iota = lax.broadcasted_iota(jnp.int32, (RB, VOCAB_PAD), 1)
    oh = (xb[:, None] == iota).astype(jnp.float32)
    h = jnp.dot(oh, embd_ref[...], preferred_element_type=jnp.float32)
    h_ref[...] = h
    hx = jnp.dot(h, w_ref[...], preferred_element_type=jnp.float32) + b_ref[...]
    hx_ref[...] = jnp.maximum(hx, 0.0)


@jax.jit
def _embed_lin(x3, embd_p, w, b):
    return pl.pallas_call(
        _embed_lin_kernel,
        grid=(NB,),
        in_specs=[
            pl.BlockSpec((1, 1, RB), lambda i: (i, 0, 0)),
            pl.BlockSpec((VOCAB_PAD, DIM), lambda i: (0, 0)),
            pl.BlockSpec((DIM, DIM), lambda i: (0, 0)),
            pl.BlockSpec((1, DIM), lambda i: (0, 0)),
        ],
        out_specs=[
            pl.BlockSpec((RB, DIM), lambda i: (i, 0)),
            pl.BlockSpec((RB, DIM), lambda i: (i, 0)),
        ],
        out_shape=[
            jax.ShapeDtypeStruct((N_NODES, DIM), jnp.float32),
            jax.ShapeDtypeStruct((N_NODES, DIM), jnp.float32),
        ],
    )(x3, embd_p, w, b)


def _layer_kernel(agg_ref, h_ref, w_ref, b_ref, hn_ref, hx_ref):
    s = agg_ref[0] + agg_ref[1] + h_ref[...]
    ss = jnp.sum(s * s, axis=1, keepdims=True)
    nrm = jnp.maximum(jnp.sqrt(ss), 1e-12)
    hn = s / nrm
    hn_ref[...] = hn
    hx = jnp.dot(hn, w_ref[...], preferred_element_type=jnp.float32) + b_ref[...]
    hx_ref[...] = jnp.maximum(hx, 0.0)


@jax.jit
def _layer(agg, h, w, b):
    return pl.pallas_call(
        _layer_kernel,
        grid=(NB,),
        in_specs=[
            pl.BlockSpec((NC, RB, DIM), lambda i: (0, i, 0)),
            pl.BlockSpec((RB, DIM), lambda i: (i, 0)),
            pl.BlockSpec((DIM, DIM), lambda i: (0, 0)),
            pl.BlockSpec((1, DIM), lambda i: (0, 0)),
        ],
        out_specs=[
            pl.BlockSpec((RB, DIM), lambda i: (i, 0)),
            pl.BlockSpec((RB, DIM), lambda i: (i, 0)),
        ],
        out_shape=[
            jax.ShapeDtypeStruct((N_NODES, DIM), jnp.float32),
            jax.ShapeDtypeStruct((N_NODES, DIM), jnp.float32),
        ],
    )(agg, h, w, b)


def _readout_kernel(seg_ref, wl_ref, bl_ref, wp_ref, bp_ref, out_ref):
    m = seg_ref[0] + seg_ref[1]
    for i in range(2):
        m = jnp.dot(m, wl_ref[i], preferred_element_type=jnp.float32)
        m = jnp.maximum(m + bl_ref[i:i + 1, :], 0.0)
    out = jnp.dot(m, wp_ref[...], preferred_element_type=jnp.float32)
    out_ref[...] = out + bp_ref[...]


@jax.jit
def _readout(seg, wl, bl, wp, bp):
    return pl.pallas_call(
        _readout_kernel,
        out_shape=jax.ShapeDtypeStruct((N_GRAPHS, 1), jnp.float32),
    )(seg, wl, bl, wp, bp)


def kernel(x, edge_index, batch, embd, W_g, b_g, W_l, b_l, W_p, b_p):
    x3 = x.astype(jnp.int32).reshape(NB, 1, RB)
    idx = edge_index.astype(jnp.int32).reshape(2, NCH, ECHUNK)
    batch = batch.astype(jnp.int32)
    embd_p = jnp.pad(embd, ((0, VOCAB_PAD - embd.shape[0]), (0, 0)))

    h, hx = _embed_lin(x3, embd_p, W_g[0], b_g[0].reshape(1, DIM))
    for m in range(2):
        agg = _edge_agg(hx, idx)
        h, hx = _layer(agg, h, W_g[m + 1], b_g[m + 1].reshape(1, DIM))
    agg = _edge_agg(hx, idx)

    seg = _segsum(agg, h, batch)
    props = _readout(seg, W_l, b_l, W_p, b_p.reshape(1, 1))
    return props.reshape(N_GRAPHS)
